# baseline (speedup 1.0000x reference)
import jax
import jax.numpy as jnp
from jax.experimental import pallas as pl
from jax.experimental.pallas import tpu as pltpu

N = 1024; E = 16384; D = 64; IN_DIM = 2; OUT_DIM = 1
B = 8; HORIZON = 12; R = 3; NB = 3; GRU_UNITS = 100; ODE_STEPS = 2


def _apply_lin(p, h):
    return h @ p["w"] + p["b"]


def _rgcn(p, h, edge_index, edge_type):
    W = jnp.einsum("rb,bij->rij", p["comp"], p["basis"])
    hr = jnp.einsum("bnd,rde->brne", h, W)
    src, dst = edge_index[0], edge_index[1]
    msg = hr[:, edge_type, src, :]
    agg = jnp.zeros_like(h).at[:, dst, :].add(msg)
    deg = jnp.zeros((N,), jnp.float32).at[dst].add(1.0)
    agg = agg / jnp.clip(deg, 1.0)[None, :, None]
    return jnp.tanh(agg + h @ p["root"] + p["bias"])


def _ode_evolve(p, h, edge_index, edge_type):
    dt = 1.0 / ODE_STEPS
    for _ in range(ODE_STEPS):
        h = h + dt * _rgcn(p, h, edge_index, edge_type)
    return h


def _gru(p, h, x):
    inp = jnp.concatenate([h, x], axis=-1)
    z = jax.nn.sigmoid(_apply_lin(p["z2"], jnp.tanh(_apply_lin(p["z1"], inp))))
    r = jax.nn.sigmoid(_apply_lin(p["r2"], jnp.tanh(_apply_lin(p["r1"], inp))))
    inp2 = jnp.concatenate([h * r, x], axis=-1)
    h_new = jnp.tanh(_apply_lin(p["h2"], jnp.tanh(_apply_lin(p["h1"], inp2))))
    return (1.0 - z) * h_new + z * h


def _out_proj_kernel(z_ref, w_ref, b_ref, o_ref):
    o_ref[...] = z_ref[...] @ w_ref[...] + b_ref[...]


def _out_proj(z_flat, w, b):
    M = z_flat.shape[0]
    BM = 8192
    return pl.pallas_call(
        _out_proj_kernel,
        grid=(M // BM,),
        in_specs=[pl.BlockSpec((BM, D), lambda i: (i, 0)),
                  pl.BlockSpec((D, OUT_DIM), lambda i: (0, 0)),
                  pl.BlockSpec((1, OUT_DIM), lambda i: (0, 0))],
        out_specs=pl.BlockSpec((BM, OUT_DIM), lambda i: (i, 0)),
        out_shape=jax.ShapeDtypeStruct((M, OUT_DIM), jnp.float32),
    )(z_flat, w, b[None, :])


def kernel(x, y, xtime, ytime, params, edge_index, edge_type):
    h = jnp.zeros((B, N, D), jnp.float32)
    for t in range(HORIZON - 1, -1, -1):
        h = _ode_evolve(params["gde1"], h, edge_index, edge_type)
        h = _gru(params["gru_enc"], h, x[t])
    data = jnp.concatenate([x, y], axis=0)
    times = jnp.concatenate([xtime, ytime], axis=0)
    z = h
    zs = []
    for t in range(data.shape[0]):
        z = _ode_evolve(params["gde2"], z, edge_index, edge_type)
        z = _gru(params["gru_dec"], z, data[t])
        zs.append(z)
    zall = jnp.stack(zs)
    pred_flat = _out_proj(zall.reshape(-1, D), params["out"]["w"], params["out"]["b"])
    preds = pred_flat.reshape(2 * HORIZON, B, N, OUT_DIM)
    id_times = jnp.argsort(times[:, 0], stable=True)
    keep = jnp.nonzero(id_times >= HORIZON, size=HORIZON)[0]
    out = preds[keep]
    return jnp.transpose(out, (1, 0, 2, 3))


# speedup vs baseline: 19.1504x; 19.1504x over previous
import functools

import jax
import jax.numpy as jnp
from jax import lax
from jax.experimental import pallas as pl
from jax.experimental.pallas import tpu as pltpu
from jax.experimental.pallas import tpu_sc as plsc

N = 1024; E = 16384; D = 64; IN_DIM = 2; OUT_DIM = 1
B = 8; HORIZON = 12; R = 3; NB = 3; GRU_UNITS = 100; ODE_STEPS = 2
DT = 1.0 / ODE_STEPS

NC, NS = 2, 16
EPW = E // (NC * NS)
BD = B * D
GW = 128
NG = BD // GW
RPW = EPW * NG
CHUNK = 128
NCHUNK = RPW // CHUNK
AGG_ROWS = N * NG
ROWS_PER_TILE = AGG_ROWS // NS


def _sc_mesh():
    return plsc.VectorSubcoreMesh(core_axis_name="c", subcore_axis_name="s",
                                  num_cores=NC, num_subcores=NS)


@functools.cache
def _sc_agg_kernel():
    return pl.kernel(
        _sc_agg_body,
        mesh=_sc_mesh(),
        out_type=jax.ShapeDtypeStruct((NC, AGG_ROWS, GW), jnp.float32),
        scratch_types=[
            pltpu.VMEM((NCHUNK, CHUNK), jnp.int32),
            pltpu.VMEM((NCHUNK, CHUNK), jnp.int32),
            pltpu.VMEM((CHUNK, GW), jnp.float32),
            pltpu.VMEM((CHUNK, GW), jnp.float32),
            pltpu.VMEM_SHARED((AGG_ROWS, GW), jnp.float32),
            pltpu.SemaphoreType.DMA,
            pltpu.SemaphoreType.DMA,
            pltpu.SemaphoreType.DMA,
        ],
    )


def _sc_agg(hr_flat, gidx, sidx, zeros_rows):
    return _sc_agg_kernel()(hr_flat, gidx, sidx, zeros_rows)


def _sc_agg_body(hr_hbm, gidx_hbm, sidx_hbm, zeros_hbm, out_hbm,
                 gidx_v, sidx_v, buf0, buf1, agg_sh, sem0, sem1, sem_s):
    c = lax.axis_index("c")
    s = lax.axis_index("s")
    pltpu.sync_copy(gidx_hbm.at[c, s], gidx_v)
    pltpu.sync_copy(sidx_hbm.at[c, s], sidx_v)
    row0 = s * ROWS_PER_TILE
    pltpu.sync_copy(zeros_hbm.at[pl.ds(row0, ROWS_PER_TILE)],
                    agg_sh.at[pl.ds(row0, ROWS_PER_TILE)])
    plsc.subcore_barrier()
    bufs = (buf0, buf1)
    sems = (sem0, sem1)
    pltpu.async_copy(hr_hbm.at[gidx_v.at[0]], bufs[0], sems[0])
    for j in range(NCHUNK):
        cur = bufs[j % 2]
        pltpu.make_async_copy(hr_hbm.at[gidx_v.at[j]], cur, sems[j % 2]).wait()
        if j + 1 < NCHUNK:
            nxt = (j + 1) % 2
            pltpu.async_copy(hr_hbm.at[gidx_v.at[j + 1]], bufs[nxt], sems[nxt])
        pltpu.async_copy(cur, agg_sh.at[sidx_v.at[j]], sem_s, add=True).wait()
    plsc.subcore_barrier()
    pltpu.sync_copy(agg_sh.at[pl.ds(row0, ROWS_PER_TILE)],
                    out_hbm.at[c, pl.ds(row0, ROWS_PER_TILE)])


DEGW = 128
DCH = 64
DNCH = EPW // DCH
DROWS_PER_TILE = N // NS


@functools.cache
def _sc_deg_kernel():
    return pl.kernel(
        _sc_deg_body,
        mesh=_sc_mesh(),
        out_type=jax.ShapeDtypeStruct((NC, N, DEGW), jnp.float32),
        scratch_types=[
            pltpu.VMEM((DNCH, DCH), jnp.int32),
            pltpu.VMEM((DCH, DEGW), jnp.float32),
            pltpu.VMEM_SHARED((N, DEGW), jnp.float32),
            pltpu.SemaphoreType.DMA,
        ],
    )


def _sc_deg(sidx, ones, zeros):
    return _sc_deg_kernel()(sidx, ones, zeros)


def _sc_deg_body(sidx_hbm, ones_hbm, zeros_hbm, out_hbm, sidx_v, ones_v,
                 deg_sh, sem_s):
    c = lax.axis_index("c")
    s = lax.axis_index("s")
    pltpu.sync_copy(sidx_hbm.at[c, s], sidx_v)
    pltpu.sync_copy(ones_hbm, ones_v)
    row0 = s * DROWS_PER_TILE
    pltpu.sync_copy(zeros_hbm.at[pl.ds(row0, DROWS_PER_TILE)],
                    deg_sh.at[pl.ds(row0, DROWS_PER_TILE)])
    plsc.subcore_barrier()
    for j in range(DNCH):
        pltpu.async_copy(ones_v, deg_sh.at[sidx_v.at[j]], sem_s, add=True).wait()
    plsc.subcore_barrier()
    pltpu.sync_copy(deg_sh.at[pl.ds(row0, DROWS_PER_TILE)],
                    out_hbm.at[c, pl.ds(row0, DROWS_PER_TILE)])


def _env_body(comp1_ref, basis1_ref, comp2_ref, basis2_ref, deg_ref,
              w1_ref, w2_ref, inv_ref):
    w1_ref[...] = comp1_ref[...] @ basis1_ref[...]
    w2_ref[...] = comp2_ref[...] @ basis2_ref[...]
    deg = deg_ref[0, :, 0:1] + deg_ref[1, :, 0:1]
    inv_ref[...] = 1.0 / jnp.maximum(deg, 1.0)


def _tc_env(comp1, basis1, comp2, basis2, deg_part):
    return pl.pallas_call(
        _env_body,
        out_shape=[jax.ShapeDtypeStruct((R, D * D), jnp.float32),
                   jax.ShapeDtypeStruct((R, D * D), jnp.float32),
                   jax.ShapeDtypeStruct((N, 1), jnp.float32)],
    )(comp1, basis1.reshape(NB, D * D), comp2, basis2.reshape(NB, D * D),
      deg_part)


def _rgcn_update(z, agg_ref, hroot_ref, inv_ref, bias_ref):
    agg = (agg_ref[0] + agg_ref[1]) * inv_ref[...]
    return z + DT * jnp.tanh(agg + hroot_ref[...] + bias_ref[...])


def _proj(z, w_ref, root_ref, hr_ref, hroot_ref):
    for r in range(R):
        hr_ref[r] = z @ w_ref[r]
    hroot_ref[...] = z @ root_ref[...]


def _postpre_body(z_ref, agg_ref, hroot_ref, inv_ref, bias_ref,
                  w_ref, root_ref, zo_ref, hr_ref, hroot_o_ref):
    z = _rgcn_update(z_ref[...], agg_ref, hroot_ref, inv_ref, bias_ref)
    zo_ref[...] = z
    _proj(z, w_ref, root_ref, hr_ref, hroot_o_ref)


BM = 2048
NBLK = (N * B) // BM

_rows = lambda i: (i, 0)
_rows3 = lambda i: (0, i, 0)
_full2 = lambda i: (0, 0)
_full3 = lambda i: (0, 0, 0)


def _tc_postpre(z, agg, hroot, inv_rep, bias, w, root):
    return pl.pallas_call(
        _postpre_body,
        grid=(NBLK,),
        in_specs=[pl.BlockSpec((BM, D), _rows),
                  pl.BlockSpec((NC, BM, D), _rows3),
                  pl.BlockSpec((BM, D), _rows),
                  pl.BlockSpec((BM, 1), _rows),
                  pl.BlockSpec((1, D), _full2),
                  pl.BlockSpec((R, D, D), _full3),
                  pl.BlockSpec((D, D), _full2)],
        out_specs=[pl.BlockSpec((BM, D), _rows),
                   pl.BlockSpec((R, BM, D), _rows3),
                   pl.BlockSpec((BM, D), _rows)],
        out_shape=[jax.ShapeDtypeStruct((N * B, D), jnp.float32),
                   jax.ShapeDtypeStruct((R, N * B, D), jnp.float32),
                   jax.ShapeDtypeStruct((N * B, D), jnp.float32)],
    )(z, agg, hroot, inv_rep, bias[None, :], w.reshape(R, D, D), root)


def _lin(h, w_ref, b_ref):
    return h @ w_ref[...] + b_ref[...]


def _gru_body(with_pred, z_ref, agg_ref, hroot_ref, inv_ref, bias_ref,
              x_ref, wz1_ref, bz1_ref, wz2_ref, bz2_ref,
              wr1_ref, br1_ref, wr2_ref, br2_ref,
              wh1_ref, bh1_ref, wh2_ref, bh2_ref,
              wn_ref, rootn_ref, ow_ref, ob_ref,
              zo_ref, hr_ref, hroot_o_ref, *maybe_pred):
    h = _rgcn_update(z_ref[...], agg_ref, hroot_ref, inv_ref, bias_ref)
    x = x_ref[...]
    def cat_lin(hh, w_ref, b_ref):
        return hh @ w_ref[0:D, :] + x @ w_ref[D:D + IN_DIM, :] + b_ref[...]

    zg = jax.nn.sigmoid(_lin(jnp.tanh(cat_lin(h, wz1_ref, bz1_ref)), wz2_ref, bz2_ref))
    rg = jax.nn.sigmoid(_lin(jnp.tanh(cat_lin(h, wr1_ref, br1_ref)), wr2_ref, br2_ref))
    hn = jnp.tanh(_lin(jnp.tanh(cat_lin(h * rg, wh1_ref, bh1_ref)), wh2_ref, bh2_ref))
    hnew = (1.0 - zg) * hn + zg * h
    zo_ref[...] = hnew
    _proj(hnew, wn_ref, rootn_ref, hr_ref, hroot_o_ref)
    if with_pred:
        maybe_pred[0][...] = hnew @ ow_ref[...] + ob_ref[...]


def _tc_gru(z, agg, hroot, inv_rep, bias, xt, gp, wn, rootn, ow, ob,
            with_pred):
    out_shape = [jax.ShapeDtypeStruct((N * B, D), jnp.float32),
                 jax.ShapeDtypeStruct((R, N * B, D), jnp.float32),
                 jax.ShapeDtypeStruct((N * B, D), jnp.float32)]
    out_specs = [pl.BlockSpec((BM, D), _rows),
                 pl.BlockSpec((R, BM, D), _rows3),
                 pl.BlockSpec((BM, D), _rows)]
    if with_pred:
        out_shape.append(jax.ShapeDtypeStruct((N * B, OUT_DIM), jnp.float32))
        out_specs.append(pl.BlockSpec((BM, OUT_DIM), _rows))
    gi = D + IN_DIM
    in_specs = ([pl.BlockSpec((BM, D), _rows),
                 pl.BlockSpec((NC, BM, D), _rows3),
                 pl.BlockSpec((BM, D), _rows),
                 pl.BlockSpec((BM, 1), _rows),
                 pl.BlockSpec((1, D), _full2),
                 pl.BlockSpec((BM, IN_DIM), _rows)]
                + [pl.BlockSpec((gi, GRU_UNITS), _full2),
                   pl.BlockSpec((1, GRU_UNITS), _full2),
                   pl.BlockSpec((GRU_UNITS, D), _full2),
                   pl.BlockSpec((1, D), _full2)] * 3
                + [pl.BlockSpec((R, D, D), _full3),
                   pl.BlockSpec((D, D), _full2),
                   pl.BlockSpec((D, OUT_DIM), _full2),
                   pl.BlockSpec((1, OUT_DIM), _full2)])
    return pl.pallas_call(
        functools.partial(_gru_body, with_pred),
        grid=(NBLK,),
        in_specs=in_specs,
        out_specs=out_specs,
        out_shape=out_shape,
    )(z, agg, hroot, inv_rep, bias[None, :], xt,
      gp["z1"]["w"], gp["z1"]["b"][None, :], gp["z2"]["w"], gp["z2"]["b"][None, :],
      gp["r1"]["w"], gp["r1"]["b"][None, :], gp["r2"]["w"], gp["r2"]["b"][None, :],
      gp["h1"]["w"], gp["h1"]["b"][None, :], gp["h2"]["w"], gp["h2"]["b"][None, :],
      wn.reshape(R, D, D), rootn, ow, ob[None, :])


def kernel(x, y, xtime, ytime, params, edge_index, edge_type):
    src, dst = edge_index[0], edge_index[1]
    groups = jnp.arange(NG, dtype=jnp.int32)
    gidx = ((edge_type * N + src)[:, None] * NG + groups).reshape(
        NC, NS, NCHUNK, CHUNK)
    sidx = (dst[:, None] * NG + groups).reshape(NC, NS, NCHUNK, CHUNK)
    sidx_deg = dst.reshape(NC, NS, DNCH, DCH)
    zeros_rows = jnp.zeros((AGG_ROWS, GW), jnp.float32)

    data = jnp.concatenate([x, y], axis=0)
    data_nb = jnp.transpose(data, (0, 2, 1, 3)).reshape(2 * HORIZON, N * B,
                                                        IN_DIM)
    times = jnp.concatenate([xtime, ytime], axis=0)

    p1, p2 = params["gde1"], params["gde2"]
    genc, gdec = params["gru_enc"], params["gru_dec"]
    ow, ob = params["out"]["w"], params["out"]["b"]

    deg_part = _sc_deg(sidx_deg, jnp.ones((DCH, DEGW), jnp.float32),
                       jnp.zeros((N, DEGW), jnp.float32))
    w1, w2, inv = _tc_env(p1["comp"], p1["basis"], p2["comp"], p2["basis"],
                          deg_part)
    inv_rep = jnp.repeat(inv.reshape(N), B).reshape(N * B, 1)

    z = jnp.zeros((N * B, D), jnp.float32)
    hr = jnp.zeros((R, N * B, D), jnp.float32)
    hroot = jnp.zeros((N * B, D), jnp.float32)
    zero_agg = jnp.zeros((NC, N * B, D), jnp.float32)

    preds = []
    for step in range(3 * HORIZON):
        enc = step < HORIZON
        w, root, bias = ((w1, p1["root"], p1["bias"]) if enc
                         else (w2, p2["root"], p2["bias"]))
        if step == 0:
            agg = zero_agg
        else:
            agg = _sc_agg(hr.reshape(R * N * NG, GW), gidx, sidx,
                          zeros_rows).reshape(NC, N * B, D)
        z, hr, hroot = _tc_postpre(z, agg, hroot, inv_rep, bias, w, root)
        agg = _sc_agg(hr.reshape(R * N * NG, GW), gidx, sidx,
                      zeros_rows).reshape(NC, N * B, D)
        nxt_enc = (step + 1) < HORIZON
        wn, rootn = (w1, p1["root"]) if nxt_enc else (w2, p2["root"])
        gp = genc if enc else gdec
        xt = data_nb[HORIZON - 1 - step] if enc else data_nb[step - HORIZON]
        outs = _tc_gru(z, agg, hroot, inv_rep, bias, xt, gp, wn, rootn,
                       ow, ob, with_pred=not enc)
        if enc:
            z, hr, hroot = outs
        else:
            z, hr, hroot, pred = outs
            preds.append(pred.reshape(N, B))

    preds = jnp.stack(preds)
    id_times = jnp.argsort(times[:, 0], stable=True)
    keep = jnp.nonzero(id_times >= HORIZON, size=HORIZON)[0]
    out = preds[keep]
    return jnp.transpose(out, (2, 0, 1))[..., None]


# speedup vs baseline: 20.5477x; 1.0730x over previous
import functools

import jax
import jax.numpy as jnp
from jax import lax
from jax.experimental import pallas as pl
from jax.experimental.pallas import tpu as pltpu
from jax.experimental.pallas import tpu_sc as plsc

N = 1024; E = 16384; D = 64; IN_DIM = 2; OUT_DIM = 1
B = 8; HORIZON = 12; R = 3; NB = 3; GRU_UNITS = 100; ODE_STEPS = 2
DT = 1.0 / ODE_STEPS

NC, NS = 2, 16
EPW = E // (NC * NS)
BD = B * D
GW = 128
NG = BD // GW
RPW = EPW * NG
NBUF = 4
CHUNK = 128
NCHUNK = RPW // CHUNK
AGG_ROWS = N * NG
ROWS_PER_TILE = AGG_ROWS // NS


def _sc_mesh():
    return plsc.VectorSubcoreMesh(core_axis_name="c", subcore_axis_name="s",
                                  num_cores=NC, num_subcores=NS)


@functools.cache
def _sc_agg_kernel():
    return pl.kernel(
        _sc_agg_body,
        mesh=_sc_mesh(),
        out_type=jax.ShapeDtypeStruct((NC, AGG_ROWS, GW), jnp.float32),
        scratch_types=[
            pltpu.VMEM((NCHUNK, CHUNK), jnp.int32),
            pltpu.VMEM((NCHUNK, CHUNK), jnp.int32),
            [pltpu.VMEM((CHUNK, GW), jnp.float32)] * NBUF,
            pltpu.VMEM_SHARED((AGG_ROWS, GW), jnp.float32),
            [pltpu.SemaphoreType.DMA] * NBUF,
            [pltpu.SemaphoreType.DMA] * NBUF,
        ],
    )


def _sc_agg(hr_flat, gidx, sidx, zeros_rows):
    return _sc_agg_kernel()(hr_flat, gidx, sidx, zeros_rows)


def _sc_agg_body(hr_hbm, gidx_hbm, sidx_hbm, zeros_hbm, out_hbm,
                 gidx_v, sidx_v, bufs, agg_sh, gsem, ssem):
    c = lax.axis_index("c")
    s = lax.axis_index("s")
    pltpu.sync_copy(gidx_hbm.at[c, s], gidx_v)
    pltpu.sync_copy(sidx_hbm.at[c, s], sidx_v)
    row0 = s * ROWS_PER_TILE
    pltpu.sync_copy(zeros_hbm.at[pl.ds(row0, ROWS_PER_TILE)],
                    agg_sh.at[pl.ds(row0, ROWS_PER_TILE)])
    plsc.subcore_barrier()

    def gather(k):
        pltpu.async_copy(hr_hbm.at[gidx_v.at[k]], bufs[k % NBUF],
                         gsem[k % NBUF])

    def gather_wait(k):
        pltpu.make_async_copy(hr_hbm.at[gidx_v.at[k]], bufs[k % NBUF],
                              gsem[k % NBUF]).wait()

    def scatter(k):
        pltpu.async_copy(bufs[k % NBUF], agg_sh.at[sidx_v.at[k]],
                         ssem[k % NBUF], add=True)

    def scatter_wait(k):
        pltpu.make_async_copy(bufs[k % NBUF], agg_sh.at[sidx_v.at[k]],
                              ssem[k % NBUF]).wait()

    gather(0)
    gather(1)
    for j in range(NCHUNK):
        if j + 2 < NCHUNK:
            if j - 2 >= 0:
                scatter_wait(j - 2)
            gather(j + 2)
        gather_wait(j)
        scatter(j)
    for j in range(max(0, NCHUNK - 4), NCHUNK):
        scatter_wait(j)
    plsc.subcore_barrier()
    pltpu.sync_copy(agg_sh.at[pl.ds(row0, ROWS_PER_TILE)],
                    out_hbm.at[c, pl.ds(row0, ROWS_PER_TILE)])


DEGW = 128
DCH = 64
DNCH = EPW // DCH
DROWS_PER_TILE = N // NS


@functools.cache
def _sc_deg_kernel():
    return pl.kernel(
        _sc_deg_body,
        mesh=_sc_mesh(),
        out_type=jax.ShapeDtypeStruct((NC, N, DEGW), jnp.float32),
        scratch_types=[
            pltpu.VMEM((DNCH, DCH), jnp.int32),
            pltpu.VMEM((DCH, DEGW), jnp.float32),
            pltpu.VMEM_SHARED((N, DEGW), jnp.float32),
            pltpu.SemaphoreType.DMA,
        ],
    )


def _sc_deg(sidx, ones, zeros):
    return _sc_deg_kernel()(sidx, ones, zeros)


def _sc_deg_body(sidx_hbm, ones_hbm, zeros_hbm, out_hbm, sidx_v, ones_v,
                 deg_sh, sem_s):
    c = lax.axis_index("c")
    s = lax.axis_index("s")
    pltpu.sync_copy(sidx_hbm.at[c, s], sidx_v)
    pltpu.sync_copy(ones_hbm, ones_v)
    row0 = s * DROWS_PER_TILE
    pltpu.sync_copy(zeros_hbm.at[pl.ds(row0, DROWS_PER_TILE)],
                    deg_sh.at[pl.ds(row0, DROWS_PER_TILE)])
    plsc.subcore_barrier()
    for j in range(DNCH):
        pltpu.async_copy(ones_v, deg_sh.at[sidx_v.at[j]], sem_s, add=True).wait()
    plsc.subcore_barrier()
    pltpu.sync_copy(deg_sh.at[pl.ds(row0, DROWS_PER_TILE)],
                    out_hbm.at[c, pl.ds(row0, DROWS_PER_TILE)])


def _env_body(comp1_ref, basis1_ref, comp2_ref, basis2_ref, deg_ref,
              w1_ref, w2_ref, inv_ref):
    w1_ref[...] = comp1_ref[...] @ basis1_ref[...]
    w2_ref[...] = comp2_ref[...] @ basis2_ref[...]
    deg = deg_ref[0, :, 0:1] + deg_ref[1, :, 0:1]
    inv_ref[...] = 1.0 / jnp.maximum(deg, 1.0)


def _tc_env(comp1, basis1, comp2, basis2, deg_part):
    return pl.pallas_call(
        _env_body,
        out_shape=[jax.ShapeDtypeStruct((R, D * D), jnp.float32),
                   jax.ShapeDtypeStruct((R, D * D), jnp.float32),
                   jax.ShapeDtypeStruct((N, 1), jnp.float32)],
    )(comp1, basis1.reshape(NB, D * D), comp2, basis2.reshape(NB, D * D),
      deg_part)


def _rgcn_update(z, agg_ref, hroot_ref, inv_ref, bias_ref):
    agg = (agg_ref[0] + agg_ref[1]) * inv_ref[...]
    return z + DT * jnp.tanh(agg + hroot_ref[...] + bias_ref[...])


def _proj(z, w_ref, root_ref, hr_ref, hroot_ref):
    for r in range(R):
        hr_ref[r] = z @ w_ref[r]
    hroot_ref[...] = z @ root_ref[...]


def _postpre_body(z_ref, agg_ref, hroot_ref, inv_ref, bias_ref,
                  w_ref, root_ref, zo_ref, hr_ref, hroot_o_ref):
    z = _rgcn_update(z_ref[...], agg_ref, hroot_ref, inv_ref, bias_ref)
    zo_ref[...] = z
    _proj(z, w_ref, root_ref, hr_ref, hroot_o_ref)


BM = 2048
NBLK = (N * B) // BM

_rows = lambda i: (i, 0)
_rows3 = lambda i: (0, i, 0)
_full2 = lambda i: (0, 0)
_full3 = lambda i: (0, 0, 0)


def _tc_postpre(z, agg, hroot, inv_rep, bias, w, root):
    return pl.pallas_call(
        _postpre_body,
        grid=(NBLK,),
        in_specs=[pl.BlockSpec((BM, D), _rows),
                  pl.BlockSpec((NC, BM, D), _rows3),
                  pl.BlockSpec((BM, D), _rows),
                  pl.BlockSpec((BM, 1), _rows),
                  pl.BlockSpec((1, D), _full2),
                  pl.BlockSpec((R, D, D), _full3),
                  pl.BlockSpec((D, D), _full2)],
        out_specs=[pl.BlockSpec((BM, D), _rows),
                   pl.BlockSpec((R, BM, D), _rows3),
                   pl.BlockSpec((BM, D), _rows)],
        out_shape=[jax.ShapeDtypeStruct((N * B, D), jnp.float32),
                   jax.ShapeDtypeStruct((R, N * B, D), jnp.float32),
                   jax.ShapeDtypeStruct((N * B, D), jnp.float32)],
    )(z, agg, hroot, inv_rep, bias[None, :], w.reshape(R, D, D), root)


def _lin(h, w_ref, b_ref):
    return h @ w_ref[...] + b_ref[...]


def _gru_body(with_pred, z_ref, agg_ref, hroot_ref, inv_ref, bias_ref,
              x_ref, wz1_ref, bz1_ref, wz2_ref, bz2_ref,
              wr1_ref, br1_ref, wr2_ref, br2_ref,
              wh1_ref, bh1_ref, wh2_ref, bh2_ref,
              wn_ref, rootn_ref, ow_ref, ob_ref,
              zo_ref, hr_ref, hroot_o_ref, *maybe_pred):
    h = _rgcn_update(z_ref[...], agg_ref, hroot_ref, inv_ref, bias_ref)
    x = x_ref[...]
    def cat_lin(hh, w_ref, b_ref):
        return hh @ w_ref[0:D, :] + x @ w_ref[D:D + IN_DIM, :] + b_ref[...]

    zg = jax.nn.sigmoid(_lin(jnp.tanh(cat_lin(h, wz1_ref, bz1_ref)), wz2_ref, bz2_ref))
    rg = jax.nn.sigmoid(_lin(jnp.tanh(cat_lin(h, wr1_ref, br1_ref)), wr2_ref, br2_ref))
    hn = jnp.tanh(_lin(jnp.tanh(cat_lin(h * rg, wh1_ref, bh1_ref)), wh2_ref, bh2_ref))
    hnew = (1.0 - zg) * hn + zg * h
    zo_ref[...] = hnew
    _proj(hnew, wn_ref, rootn_ref, hr_ref, hroot_o_ref)
    if with_pred:
        maybe_pred[0][...] = hnew @ ow_ref[...] + ob_ref[...]


def _tc_gru(z, agg, hroot, inv_rep, bias, xt, gp, wn, rootn, ow, ob,
            with_pred):
    out_shape = [jax.ShapeDtypeStruct((N * B, D), jnp.float32),
                 jax.ShapeDtypeStruct((R, N * B, D), jnp.float32),
                 jax.ShapeDtypeStruct((N * B, D), jnp.float32)]
    out_specs = [pl.BlockSpec((BM, D), _rows),
                 pl.BlockSpec((R, BM, D), _rows3),
                 pl.BlockSpec((BM, D), _rows)]
    if with_pred:
        out_shape.append(jax.ShapeDtypeStruct((N * B, OUT_DIM), jnp.float32))
        out_specs.append(pl.BlockSpec((BM, OUT_DIM), _rows))
    gi = D + IN_DIM
    in_specs = ([pl.BlockSpec((BM, D), _rows),
                 pl.BlockSpec((NC, BM, D), _rows3),
                 pl.BlockSpec((BM, D), _rows),
                 pl.BlockSpec((BM, 1), _rows),
                 pl.BlockSpec((1, D), _full2),
                 pl.BlockSpec((BM, IN_DIM), _rows)]
                + [pl.BlockSpec((gi, GRU_UNITS), _full2),
                   pl.BlockSpec((1, GRU_UNITS), _full2),
                   pl.BlockSpec((GRU_UNITS, D), _full2),
                   pl.BlockSpec((1, D), _full2)] * 3
                + [pl.BlockSpec((R, D, D), _full3),
                   pl.BlockSpec((D, D), _full2),
                   pl.BlockSpec((D, OUT_DIM), _full2),
                   pl.BlockSpec((1, OUT_DIM), _full2)])
    return pl.pallas_call(
        functools.partial(_gru_body, with_pred),
        grid=(NBLK,),
        in_specs=in_specs,
        out_specs=out_specs,
        out_shape=out_shape,
    )(z, agg, hroot, inv_rep, bias[None, :], xt,
      gp["z1"]["w"], gp["z1"]["b"][None, :], gp["z2"]["w"], gp["z2"]["b"][None, :],
      gp["r1"]["w"], gp["r1"]["b"][None, :], gp["r2"]["w"], gp["r2"]["b"][None, :],
      gp["h1"]["w"], gp["h1"]["b"][None, :], gp["h2"]["w"], gp["h2"]["b"][None, :],
      wn.reshape(R, D, D), rootn, ow, ob[None, :])


def kernel(x, y, xtime, ytime, params, edge_index, edge_type):
    src, dst = edge_index[0], edge_index[1]
    groups = jnp.arange(NG, dtype=jnp.int32)
    gidx = ((edge_type * N + src)[:, None] * NG + groups).reshape(
        NC, NS, NCHUNK, CHUNK)
    sidx = (dst[:, None] * NG + groups).reshape(NC, NS, NCHUNK, CHUNK)
    sidx_deg = dst.reshape(NC, NS, DNCH, DCH)
    zeros_rows = jnp.zeros((AGG_ROWS, GW), jnp.float32)

    data = jnp.concatenate([x, y], axis=0)
    data_nb = jnp.transpose(data, (0, 2, 1, 3)).reshape(2 * HORIZON, N * B,
                                                        IN_DIM)
    times = jnp.concatenate([xtime, ytime], axis=0)

    p1, p2 = params["gde1"], params["gde2"]
    genc, gdec = params["gru_enc"], params["gru_dec"]
    ow, ob = params["out"]["w"], params["out"]["b"]

    deg_part = _sc_deg(sidx_deg, jnp.ones((DCH, DEGW), jnp.float32),
                       jnp.zeros((N, DEGW), jnp.float32))
    w1, w2, inv = _tc_env(p1["comp"], p1["basis"], p2["comp"], p2["basis"],
                          deg_part)
    inv_rep = jnp.repeat(inv.reshape(N), B).reshape(N * B, 1)

    z = jnp.zeros((N * B, D), jnp.float32)
    hr = jnp.zeros((R, N * B, D), jnp.float32)
    hroot = jnp.zeros((N * B, D), jnp.float32)
    zero_agg = jnp.zeros((NC, N * B, D), jnp.float32)

    preds = []
    for step in range(3 * HORIZON):
        enc = step < HORIZON
        w, root, bias = ((w1, p1["root"], p1["bias"]) if enc
                         else (w2, p2["root"], p2["bias"]))
        if step == 0:
            agg = zero_agg
        else:
            agg = _sc_agg(hr.reshape(R * N * NG, GW), gidx, sidx,
                          zeros_rows).reshape(NC, N * B, D)
        z, hr, hroot = _tc_postpre(z, agg, hroot, inv_rep, bias, w, root)
        agg = _sc_agg(hr.reshape(R * N * NG, GW), gidx, sidx,
                      zeros_rows).reshape(NC, N * B, D)
        nxt_enc = (step + 1) < HORIZON
        wn, rootn = (w1, p1["root"]) if nxt_enc else (w2, p2["root"])
        gp = genc if enc else gdec
        xt = data_nb[HORIZON - 1 - step] if enc else data_nb[step - HORIZON]
        outs = _tc_gru(z, agg, hroot, inv_rep, bias, xt, gp, wn, rootn,
                       ow, ob, with_pred=not enc)
        if enc:
            z, hr, hroot = outs
        else:
            z, hr, hroot, pred = outs
            preds.append(pred.reshape(N, B))

    preds = jnp.stack(preds)
    id_times = jnp.argsort(times[:, 0], stable=True)
    keep = jnp.nonzero(id_times >= HORIZON, size=HORIZON)[0]
    out = preds[keep]
    return jnp.transpose(out, (2, 0, 1))[..., None]


# speedup vs baseline: 21.3213x; 1.0377x over previous
import functools

import jax
import jax.numpy as jnp
from jax import lax
from jax.experimental import pallas as pl
from jax.experimental.pallas import tpu as pltpu
from jax.experimental.pallas import tpu_sc as plsc

N = 1024; E = 16384; D = 64; IN_DIM = 2; OUT_DIM = 1
B = 8; HORIZON = 12; R = 3; NB = 3; GRU_UNITS = 100; ODE_STEPS = 2
DT = 1.0 / ODE_STEPS

NC, NS = 2, 16
EPW = E // (NC * NS)
BD = B * D
GW = 128
NG = BD // GW
RPW = EPW * NG
NBUF = 5
PREF = 3
CHUNK = 128
NCHUNK = RPW // CHUNK
AGG_ROWS = N * NG
ROWS_PER_TILE = AGG_ROWS // NS


def _sc_mesh():
    return plsc.VectorSubcoreMesh(core_axis_name="c", subcore_axis_name="s",
                                  num_cores=NC, num_subcores=NS)


@functools.cache
def _sc_agg_kernel():
    return pl.kernel(
        _sc_agg_body,
        mesh=_sc_mesh(),
        out_type=jax.ShapeDtypeStruct((NC, AGG_ROWS, GW), jnp.float32),
        scratch_types=[
            pltpu.VMEM((NCHUNK, CHUNK), jnp.int32),
            pltpu.VMEM((NCHUNK, CHUNK), jnp.int32),
            [pltpu.VMEM((CHUNK, GW), jnp.float32)] * NBUF,
            pltpu.VMEM_SHARED((AGG_ROWS, GW), jnp.float32),
            [pltpu.SemaphoreType.DMA] * NBUF,
            [pltpu.SemaphoreType.DMA] * NBUF,
        ],
    )


def _sc_agg(hr_flat, gidx, sidx, zeros_rows):
    return _sc_agg_kernel()(hr_flat, gidx, sidx, zeros_rows)


def _sc_agg_body(hr_hbm, gidx_hbm, sidx_hbm, zeros_hbm, out_hbm,
                 gidx_v, sidx_v, bufs, agg_sh, gsem, ssem):
    c = lax.axis_index("c")
    s = lax.axis_index("s")
    pltpu.sync_copy(gidx_hbm.at[c, s], gidx_v)
    pltpu.sync_copy(sidx_hbm.at[c, s], sidx_v)

    def gather(k):
        pltpu.async_copy(hr_hbm.at[gidx_v.at[k]], bufs[k % NBUF],
                         gsem[k % NBUF])

    def gather_wait(k):
        pltpu.make_async_copy(hr_hbm.at[gidx_v.at[k]], bufs[k % NBUF],
                              gsem[k % NBUF]).wait()

    def scatter(k):
        pltpu.async_copy(bufs[k % NBUF], agg_sh.at[sidx_v.at[k]],
                         ssem[k % NBUF], add=True)

    def scatter_wait(k):
        pltpu.make_async_copy(bufs[k % NBUF], agg_sh.at[sidx_v.at[k]],
                              ssem[k % NBUF]).wait()

    for k in range(PREF):
        gather(k)
    row0 = s * ROWS_PER_TILE
    pltpu.sync_copy(zeros_hbm.at[pl.ds(row0, ROWS_PER_TILE)],
                    agg_sh.at[pl.ds(row0, ROWS_PER_TILE)])
    plsc.subcore_barrier()
    waited = -1
    for j in range(NCHUNK):
        if j + PREF < NCHUNK:
            if j - (NBUF - PREF) >= 0:
                waited = j - (NBUF - PREF)
                scatter_wait(waited)
            gather(j + PREF)
        gather_wait(j)
        scatter(j)
    for j in range(waited + 1, NCHUNK):
        scatter_wait(j)
    plsc.subcore_barrier()
    pltpu.sync_copy(agg_sh.at[pl.ds(row0, ROWS_PER_TILE)],
                    out_hbm.at[c, pl.ds(row0, ROWS_PER_TILE)])


DEGW = 128
DCH = 64
DNCH = EPW // DCH
DROWS_PER_TILE = N // NS


@functools.cache
def _sc_deg_kernel():
    return pl.kernel(
        _sc_deg_body,
        mesh=_sc_mesh(),
        out_type=jax.ShapeDtypeStruct((NC, N, DEGW), jnp.float32),
        scratch_types=[
            pltpu.VMEM((DNCH, DCH), jnp.int32),
            pltpu.VMEM((DCH, DEGW), jnp.float32),
            pltpu.VMEM_SHARED((N, DEGW), jnp.float32),
            pltpu.SemaphoreType.DMA,
        ],
    )


def _sc_deg(sidx, ones, zeros):
    return _sc_deg_kernel()(sidx, ones, zeros)


def _sc_deg_body(sidx_hbm, ones_hbm, zeros_hbm, out_hbm, sidx_v, ones_v,
                 deg_sh, sem_s):
    c = lax.axis_index("c")
    s = lax.axis_index("s")
    pltpu.sync_copy(sidx_hbm.at[c, s], sidx_v)
    pltpu.sync_copy(ones_hbm, ones_v)
    row0 = s * DROWS_PER_TILE
    pltpu.sync_copy(zeros_hbm.at[pl.ds(row0, DROWS_PER_TILE)],
                    deg_sh.at[pl.ds(row0, DROWS_PER_TILE)])
    plsc.subcore_barrier()
    for j in range(DNCH):
        pltpu.async_copy(ones_v, deg_sh.at[sidx_v.at[j]], sem_s, add=True).wait()
    plsc.subcore_barrier()
    pltpu.sync_copy(deg_sh.at[pl.ds(row0, DROWS_PER_TILE)],
                    out_hbm.at[c, pl.ds(row0, DROWS_PER_TILE)])


def _env_body(comp1_ref, basis1_ref, comp2_ref, basis2_ref, deg_ref,
              w1_ref, w2_ref, inv_ref):
    w1_ref[...] = comp1_ref[...] @ basis1_ref[...]
    w2_ref[...] = comp2_ref[...] @ basis2_ref[...]
    deg = deg_ref[0, :, 0:1] + deg_ref[1, :, 0:1]
    inv_ref[...] = 1.0 / jnp.maximum(deg, 1.0)


def _tc_env(comp1, basis1, comp2, basis2, deg_part):
    return pl.pallas_call(
        _env_body,
        out_shape=[jax.ShapeDtypeStruct((R, D * D), jnp.float32),
                   jax.ShapeDtypeStruct((R, D * D), jnp.float32),
                   jax.ShapeDtypeStruct((N, 1), jnp.float32)],
    )(comp1, basis1.reshape(NB, D * D), comp2, basis2.reshape(NB, D * D),
      deg_part)


def _rgcn_update(z, agg_ref, hroot_ref, inv_ref, bias_ref):
    agg = (agg_ref[0] + agg_ref[1]) * inv_ref[...]
    return z + DT * jnp.tanh(agg + hroot_ref[...] + bias_ref[...])


def _proj(z, w_ref, root_ref, hr_ref, hroot_ref):
    for r in range(R):
        hr_ref[r] = z @ w_ref[r]
    hroot_ref[...] = z @ root_ref[...]


def _postpre_body(z_ref, agg_ref, hroot_ref, inv_ref, bias_ref,
                  w_ref, root_ref, zo_ref, hr_ref, hroot_o_ref):
    z = _rgcn_update(z_ref[...], agg_ref, hroot_ref, inv_ref, bias_ref)
    zo_ref[...] = z
    _proj(z, w_ref, root_ref, hr_ref, hroot_o_ref)


BM = 4096
NBLK = (N * B) // BM

_rows = lambda i: (i, 0)
_rows3 = lambda i: (0, i, 0)
_full2 = lambda i: (0, 0)
_full3 = lambda i: (0, 0, 0)


def _tc_postpre(z, agg, hroot, inv_rep, bias, w, root):
    return pl.pallas_call(
        _postpre_body,
        grid=(NBLK,),
        in_specs=[pl.BlockSpec((BM, D), _rows),
                  pl.BlockSpec((NC, BM, D), _rows3),
                  pl.BlockSpec((BM, D), _rows),
                  pl.BlockSpec((BM, 1), _rows),
                  pl.BlockSpec((1, D), _full2),
                  pl.BlockSpec((R, D, D), _full3),
                  pl.BlockSpec((D, D), _full2)],
        out_specs=[pl.BlockSpec((BM, D), _rows),
                   pl.BlockSpec((R, BM, D), _rows3),
                   pl.BlockSpec((BM, D), _rows)],
        out_shape=[jax.ShapeDtypeStruct((N * B, D), jnp.float32),
                   jax.ShapeDtypeStruct((R, N * B, D), jnp.float32),
                   jax.ShapeDtypeStruct((N * B, D), jnp.float32)],
    )(z, agg, hroot, inv_rep, bias[None, :], w.reshape(R, D, D), root)


def _lin(h, w_ref, b_ref):
    return h @ w_ref[...] + b_ref[...]


def _gru_body(with_pred, z_ref, agg_ref, hroot_ref, inv_ref, bias_ref,
              x_ref, wz1_ref, bz1_ref, wz2_ref, bz2_ref,
              wr1_ref, br1_ref, wr2_ref, br2_ref,
              wh1_ref, bh1_ref, wh2_ref, bh2_ref,
              wn_ref, rootn_ref, ow_ref, ob_ref,
              zo_ref, hr_ref, hroot_o_ref, *maybe_pred):
    h = _rgcn_update(z_ref[...], agg_ref, hroot_ref, inv_ref, bias_ref)
    x = x_ref[...]
    def cat_lin(hh, w_ref, b_ref):
        return hh @ w_ref[0:D, :] + x @ w_ref[D:D + IN_DIM, :] + b_ref[...]

    zg = jax.nn.sigmoid(_lin(jnp.tanh(cat_lin(h, wz1_ref, bz1_ref)), wz2_ref, bz2_ref))
    rg = jax.nn.sigmoid(_lin(jnp.tanh(cat_lin(h, wr1_ref, br1_ref)), wr2_ref, br2_ref))
    hn = jnp.tanh(_lin(jnp.tanh(cat_lin(h * rg, wh1_ref, bh1_ref)), wh2_ref, bh2_ref))
    hnew = (1.0 - zg) * hn + zg * h
    zo_ref[...] = hnew
    _proj(hnew, wn_ref, rootn_ref, hr_ref, hroot_o_ref)
    if with_pred:
        maybe_pred[0][...] = hnew @ ow_ref[...] + ob_ref[...]


def _tc_gru(z, agg, hroot, inv_rep, bias, xt, gp, wn, rootn, ow, ob,
            with_pred):
    out_shape = [jax.ShapeDtypeStruct((N * B, D), jnp.float32),
                 jax.ShapeDtypeStruct((R, N * B, D), jnp.float32),
                 jax.ShapeDtypeStruct((N * B, D), jnp.float32)]
    out_specs = [pl.BlockSpec((BM, D), _rows),
                 pl.BlockSpec((R, BM, D), _rows3),
                 pl.BlockSpec((BM, D), _rows)]
    if with_pred:
        out_shape.append(jax.ShapeDtypeStruct((N * B, OUT_DIM), jnp.float32))
        out_specs.append(pl.BlockSpec((BM, OUT_DIM), _rows))
    gi = D + IN_DIM
    in_specs = ([pl.BlockSpec((BM, D), _rows),
                 pl.BlockSpec((NC, BM, D), _rows3),
                 pl.BlockSpec((BM, D), _rows),
                 pl.BlockSpec((BM, 1), _rows),
                 pl.BlockSpec((1, D), _full2),
                 pl.BlockSpec((BM, IN_DIM), _rows)]
                + [pl.BlockSpec((gi, GRU_UNITS), _full2),
                   pl.BlockSpec((1, GRU_UNITS), _full2),
                   pl.BlockSpec((GRU_UNITS, D), _full2),
                   pl.BlockSpec((1, D), _full2)] * 3
                + [pl.BlockSpec((R, D, D), _full3),
                   pl.BlockSpec((D, D), _full2),
                   pl.BlockSpec((D, OUT_DIM), _full2),
                   pl.BlockSpec((1, OUT_DIM), _full2)])
    return pl.pallas_call(
        functools.partial(_gru_body, with_pred),
        grid=(NBLK,),
        in_specs=in_specs,
        out_specs=out_specs,
        out_shape=out_shape,
    )(z, agg, hroot, inv_rep, bias[None, :], xt,
      gp["z1"]["w"], gp["z1"]["b"][None, :], gp["z2"]["w"], gp["z2"]["b"][None, :],
      gp["r1"]["w"], gp["r1"]["b"][None, :], gp["r2"]["w"], gp["r2"]["b"][None, :],
      gp["h1"]["w"], gp["h1"]["b"][None, :], gp["h2"]["w"], gp["h2"]["b"][None, :],
      wn.reshape(R, D, D), rootn, ow, ob[None, :])


def kernel(x, y, xtime, ytime, params, edge_index, edge_type):
    src, dst = edge_index[0], edge_index[1]
    groups = jnp.arange(NG, dtype=jnp.int32)
    gidx = ((edge_type * N + src)[:, None] * NG + groups).reshape(
        NC, NS, NCHUNK, CHUNK)
    sidx = (dst[:, None] * NG + groups).reshape(NC, NS, NCHUNK, CHUNK)
    sidx_deg = dst.reshape(NC, NS, DNCH, DCH)
    zeros_rows = jnp.zeros((AGG_ROWS, GW), jnp.float32)

    data = jnp.concatenate([x, y], axis=0)
    data_nb = jnp.transpose(data, (0, 2, 1, 3)).reshape(2 * HORIZON, N * B,
                                                        IN_DIM)
    times = jnp.concatenate([xtime, ytime], axis=0)

    p1, p2 = params["gde1"], params["gde2"]
    genc, gdec = params["gru_enc"], params["gru_dec"]
    ow, ob = params["out"]["w"], params["out"]["b"]

    deg_part = _sc_deg(sidx_deg, jnp.ones((DCH, DEGW), jnp.float32),
                       jnp.zeros((N, DEGW), jnp.float32))
    w1, w2, inv = _tc_env(p1["comp"], p1["basis"], p2["comp"], p2["basis"],
                          deg_part)
    inv_rep = jnp.repeat(inv.reshape(N), B).reshape(N * B, 1)

    z = jnp.zeros((N * B, D), jnp.float32)
    hr = jnp.zeros((R, N * B, D), jnp.float32)
    hroot = jnp.zeros((N * B, D), jnp.float32)
    zero_agg = jnp.zeros((NC, N * B, D), jnp.float32)

    preds = []
    for step in range(3 * HORIZON):
        enc = step < HORIZON
        w, root, bias = ((w1, p1["root"], p1["bias"]) if enc
                         else (w2, p2["root"], p2["bias"]))
        if step == 0:
            agg = zero_agg
        else:
            agg = _sc_agg(hr.reshape(R * N * NG, GW), gidx, sidx,
                          zeros_rows).reshape(NC, N * B, D)
        z, hr, hroot = _tc_postpre(z, agg, hroot, inv_rep, bias, w, root)
        agg = _sc_agg(hr.reshape(R * N * NG, GW), gidx, sidx,
                      zeros_rows).reshape(NC, N * B, D)
        nxt_enc = (step + 1) < HORIZON
        wn, rootn = (w1, p1["root"]) if nxt_enc else (w2, p2["root"])
        gp = genc if enc else gdec
        xt = data_nb[HORIZON - 1 - step] if enc else data_nb[step - HORIZON]
        outs = _tc_gru(z, agg, hroot, inv_rep, bias, xt, gp, wn, rootn,
                       ow, ob, with_pred=not enc)
        if enc:
            z, hr, hroot = outs
        else:
            z, hr, hroot, pred = outs
            preds.append(pred.reshape(N, B))

    preds = jnp.stack(preds)
    id_times = jnp.argsort(times[:, 0], stable=True)
    keep = jnp.nonzero(id_times >= HORIZON, size=HORIZON)[0]
    out = preds[keep]
    return jnp.transpose(out, (2, 0, 1))[..., None]


# speedup vs baseline: 33.2380x; 1.5589x over previous
import functools

import jax
import jax.numpy as jnp
from jax import lax
from jax.experimental import pallas as pl
from jax.experimental.pallas import tpu as pltpu
from jax.experimental.pallas import tpu_sc as plsc

N = 1024; E = 16384; D = 64; IN_DIM = 2; OUT_DIM = 1
B = 8; HORIZON = 12; R = 3; NB = 3; GRU_UNITS = 100; ODE_STEPS = 2
DT = 1.0 / ODE_STEPS

NC, NS = 2, 16
EPW = E // (NC * NS)
BD = B * D
GW = 128
NG = BD // GW
RPW = EPW * NG
NBUF = 5
PREF = 3
CHUNK = 128
NCHUNK = RPW // CHUNK
AGG_ROWS = N * NG
ROWS_PER_TILE = AGG_ROWS // NS


def _sc_mesh():
    return plsc.VectorSubcoreMesh(core_axis_name="c", subcore_axis_name="s",
                                  num_cores=NC, num_subcores=NS)


@functools.cache
def _sc_agg_kernel():
    return pl.kernel(
        _sc_agg_body,
        mesh=_sc_mesh(),
        out_type=jax.ShapeDtypeStruct((NC, AGG_ROWS, GW), jnp.float32),
        scratch_types=[
            pltpu.VMEM((NCHUNK, CHUNK), jnp.int32),
            pltpu.VMEM((NCHUNK, CHUNK), jnp.int32),
            [pltpu.VMEM((CHUNK, GW), jnp.float32)] * NBUF,
            pltpu.VMEM_SHARED((AGG_ROWS, GW), jnp.float32),
            [pltpu.SemaphoreType.DMA] * NBUF,
            [pltpu.SemaphoreType.DMA] * NBUF,
        ],
    )


def _sc_agg(hr_flat, gidx, sidx, zeros_rows):
    return _sc_agg_kernel()(hr_flat, gidx, sidx, zeros_rows)


def _sc_agg_body(hr_hbm, gidx_hbm, sidx_hbm, zeros_hbm, out_hbm,
                 gidx_v, sidx_v, bufs, agg_sh, gsem, ssem):
    c = lax.axis_index("c")
    s = lax.axis_index("s")
    pltpu.sync_copy(gidx_hbm.at[c, s], gidx_v)
    pltpu.sync_copy(sidx_hbm.at[c, s], sidx_v)

    def gather(k):
        pltpu.async_copy(hr_hbm.at[gidx_v.at[k]], bufs[k % NBUF],
                         gsem[k % NBUF])

    def gather_wait(k):
        pltpu.make_async_copy(hr_hbm.at[gidx_v.at[k]], bufs[k % NBUF],
                              gsem[k % NBUF]).wait()

    def scatter(k):
        pltpu.async_copy(bufs[k % NBUF], agg_sh.at[sidx_v.at[k]],
                         ssem[k % NBUF], add=True)

    def scatter_wait(k):
        pltpu.make_async_copy(bufs[k % NBUF], agg_sh.at[sidx_v.at[k]],
                              ssem[k % NBUF]).wait()

    for k in range(PREF):
        gather(k)
    row0 = s * ROWS_PER_TILE
    pltpu.sync_copy(zeros_hbm.at[pl.ds(row0, ROWS_PER_TILE)],
                    agg_sh.at[pl.ds(row0, ROWS_PER_TILE)])
    plsc.subcore_barrier()
    waited = -1
    for j in range(NCHUNK):
        if j + PREF < NCHUNK:
            if j - (NBUF - PREF) >= 0:
                waited = j - (NBUF - PREF)
                scatter_wait(waited)
            gather(j + PREF)
        gather_wait(j)
        scatter(j)
    for j in range(waited + 1, NCHUNK):
        scatter_wait(j)
    plsc.subcore_barrier()
    pltpu.sync_copy(agg_sh.at[pl.ds(row0, ROWS_PER_TILE)],
                    out_hbm.at[c, pl.ds(row0, ROWS_PER_TILE)])


DEGW = 128
DCH = 64
DNCH = EPW // DCH
DROWS_PER_TILE = N // NS


@functools.cache
def _sc_deg_kernel():
    return pl.kernel(
        _sc_deg_body,
        mesh=_sc_mesh(),
        out_type=jax.ShapeDtypeStruct((NC, N, DEGW), jnp.float32),
        scratch_types=[
            pltpu.VMEM((DNCH, DCH), jnp.int32),
            pltpu.VMEM((DCH, DEGW), jnp.float32),
            pltpu.VMEM_SHARED((N, DEGW), jnp.float32),
            pltpu.SemaphoreType.DMA,
        ],
    )


def _sc_deg(sidx, ones, zeros):
    return _sc_deg_kernel()(sidx, ones, zeros)


def _sc_deg_body(sidx_hbm, ones_hbm, zeros_hbm, out_hbm, sidx_v, ones_v,
                 deg_sh, sem_s):
    c = lax.axis_index("c")
    s = lax.axis_index("s")
    pltpu.sync_copy(sidx_hbm.at[c, s], sidx_v)
    pltpu.sync_copy(ones_hbm, ones_v)
    row0 = s * DROWS_PER_TILE
    pltpu.sync_copy(zeros_hbm.at[pl.ds(row0, DROWS_PER_TILE)],
                    deg_sh.at[pl.ds(row0, DROWS_PER_TILE)])
    plsc.subcore_barrier()
    for j in range(DNCH):
        pltpu.async_copy(ones_v, deg_sh.at[sidx_v.at[j]], sem_s, add=True).wait()
    plsc.subcore_barrier()
    pltpu.sync_copy(deg_sh.at[pl.ds(row0, DROWS_PER_TILE)],
                    out_hbm.at[c, pl.ds(row0, DROWS_PER_TILE)])


def _env_body(comp1_ref, basis1_ref, comp2_ref, basis2_ref, deg_ref,
              w1_ref, w2_ref, inv_ref):
    w1_ref[...] = comp1_ref[...] @ basis1_ref[...]
    w2_ref[...] = comp2_ref[...] @ basis2_ref[...]
    deg = deg_ref[0, :, 0:1] + deg_ref[1, :, 0:1]
    inv_ref[...] = 1.0 / jnp.maximum(deg, 1.0)


def _tc_env(comp1, basis1, comp2, basis2, deg_part):
    return pl.pallas_call(
        _env_body,
        out_shape=[jax.ShapeDtypeStruct((R, D * D), jnp.float32),
                   jax.ShapeDtypeStruct((R, D * D), jnp.float32),
                   jax.ShapeDtypeStruct((N, 1), jnp.float32)],
    )(comp1, basis1.reshape(NB, D * D), comp2, basis2.reshape(NB, D * D),
      deg_part)


SROWS = N * NG
SW = 2 * D
G2 = 2 * GRU_UNITS
X2 = 2 * IN_DIM
P2 = 2 * OUT_DIM


def _rgcn_update(z, agg_ref, hroot_ref, inv_ref, bias_ref):
    agg = (agg_ref[0] + agg_ref[1]) * inv_ref[...]
    return z + DT * jnp.tanh(agg + hroot_ref[...] + bias_ref[...])


def _proj(z, w_ref, root_ref, hr_ref, hroot_ref):
    for r in range(R):
        hr_ref[r] = z @ w_ref[r]
    hroot_ref[...] = z @ root_ref[...]


def _postpre_body(z_ref, agg_ref, hroot_ref, inv_ref, bias_ref,
                  w_ref, root_ref, zo_ref, hr_ref, hroot_o_ref):
    z = _rgcn_update(z_ref[...], agg_ref, hroot_ref, inv_ref, bias_ref)
    zo_ref[...] = z
    _proj(z, w_ref, root_ref, hr_ref, hroot_o_ref)


BM = 2048
NBLK = SROWS // BM

_rows = lambda i: (i, 0)
_rows3 = lambda i: (0, i, 0)
_full2 = lambda i: (0, 0)
_full3 = lambda i: (0, 0, 0)


def _tc_postpre(z, agg, hroot, inv_rep, bias2, w2, root2):
    return pl.pallas_call(
        _postpre_body,
        grid=(NBLK,),
        in_specs=[pl.BlockSpec((BM, SW), _rows),
                  pl.BlockSpec((NC, BM, SW), _rows3),
                  pl.BlockSpec((BM, SW), _rows),
                  pl.BlockSpec((BM, 1), _rows),
                  pl.BlockSpec((1, SW), _full2),
                  pl.BlockSpec((R, SW, SW), _full3),
                  pl.BlockSpec((SW, SW), _full2)],
        out_specs=[pl.BlockSpec((BM, SW), _rows),
                   pl.BlockSpec((R, BM, SW), _rows3),
                   pl.BlockSpec((BM, SW), _rows)],
        out_shape=[jax.ShapeDtypeStruct((SROWS, SW), jnp.float32),
                   jax.ShapeDtypeStruct((R, SROWS, SW), jnp.float32),
                   jax.ShapeDtypeStruct((SROWS, SW), jnp.float32)],
    )(z, agg, hroot, inv_rep, bias2, w2, root2)


def _lin(h, w_ref, b_ref):
    return h @ w_ref[...] + b_ref[...]


def _gru_body(with_pred, z_ref, agg_ref, hroot_ref, inv_ref, bias_ref,
              x_ref, wz1h_ref, wz1x_ref, bz1_ref, wz2_ref, bz2_ref,
              wr1h_ref, wr1x_ref, br1_ref, wr2_ref, br2_ref,
              wh1h_ref, wh1x_ref, bh1_ref, wh2_ref, bh2_ref,
              wn_ref, rootn_ref, ow_ref, ob_ref,
              zo_ref, hr_ref, hroot_o_ref, *maybe_pred):
    h = _rgcn_update(z_ref[...], agg_ref, hroot_ref, inv_ref, bias_ref)
    x = x_ref[...]
    def cat_lin(hh, wh_ref, wx_ref, b_ref):
        return hh @ wh_ref[...] + x @ wx_ref[...] + b_ref[...]

    zg = jax.nn.sigmoid(_lin(jnp.tanh(cat_lin(h, wz1h_ref, wz1x_ref, bz1_ref)),
                             wz2_ref, bz2_ref))
    rg = jax.nn.sigmoid(_lin(jnp.tanh(cat_lin(h, wr1h_ref, wr1x_ref, br1_ref)),
                             wr2_ref, br2_ref))
    hn = jnp.tanh(_lin(jnp.tanh(cat_lin(h * rg, wh1h_ref, wh1x_ref, bh1_ref)),
                       wh2_ref, bh2_ref))
    hnew = (1.0 - zg) * hn + zg * h
    zo_ref[...] = hnew
    _proj(hnew, wn_ref, rootn_ref, hr_ref, hroot_o_ref)
    if with_pred:
        maybe_pred[0][...] = hnew @ ow_ref[...] + ob_ref[...]


def _tc_gru(z, agg, hroot, inv_rep, bias2, xt, gp2, wn2, rootn2, ow2, ob2,
            with_pred):
    out_shape = [jax.ShapeDtypeStruct((SROWS, SW), jnp.float32),
                 jax.ShapeDtypeStruct((R, SROWS, SW), jnp.float32),
                 jax.ShapeDtypeStruct((SROWS, SW), jnp.float32)]
    out_specs = [pl.BlockSpec((BM, SW), _rows),
                 pl.BlockSpec((R, BM, SW), _rows3),
                 pl.BlockSpec((BM, SW), _rows)]
    if with_pred:
        out_shape.append(jax.ShapeDtypeStruct((SROWS, P2), jnp.float32))
        out_specs.append(pl.BlockSpec((BM, P2), _rows))
    in_specs = ([pl.BlockSpec((BM, SW), _rows),
                 pl.BlockSpec((NC, BM, SW), _rows3),
                 pl.BlockSpec((BM, SW), _rows),
                 pl.BlockSpec((BM, 1), _rows),
                 pl.BlockSpec((1, SW), _full2),
                 pl.BlockSpec((BM, X2), _rows)]
                + [pl.BlockSpec((SW, G2), _full2),
                   pl.BlockSpec((X2, G2), _full2),
                   pl.BlockSpec((1, G2), _full2),
                   pl.BlockSpec((G2, SW), _full2),
                   pl.BlockSpec((1, SW), _full2)] * 3
                + [pl.BlockSpec((R, SW, SW), _full3),
                   pl.BlockSpec((SW, SW), _full2),
                   pl.BlockSpec((SW, P2), _full2),
                   pl.BlockSpec((1, P2), _full2)])
    return pl.pallas_call(
        functools.partial(_gru_body, with_pred),
        grid=(NBLK,),
        in_specs=in_specs,
        out_specs=out_specs,
        out_shape=out_shape,
    )(z, agg, hroot, inv_rep, bias2, xt,
      *gp2["z"], *gp2["r"], *gp2["h"],
      wn2, rootn2, ow2, ob2)


def kernel(x, y, xtime, ytime, params, edge_index, edge_type):
    src, dst = edge_index[0], edge_index[1]
    groups = jnp.arange(NG, dtype=jnp.int32)
    gidx = ((edge_type * N + src)[:, None] * NG + groups).reshape(
        NC, NS, NCHUNK, CHUNK)
    sidx = (dst[:, None] * NG + groups).reshape(NC, NS, NCHUNK, CHUNK)
    sidx_deg = dst.reshape(NC, NS, DNCH, DCH)
    zeros_rows = jnp.zeros((AGG_ROWS, GW), jnp.float32)

    data = jnp.concatenate([x, y], axis=0)
    data_nb = jnp.transpose(data, (0, 2, 1, 3)).reshape(
        2 * HORIZON, N, NG, 2 * IN_DIM).reshape(2 * HORIZON, SROWS, X2)
    times = jnp.concatenate([xtime, ytime], axis=0)

    def dbl(w):
        din, dout = w.shape
        z = jnp.zeros((2 * din, 2 * dout), w.dtype)
        return z.at[:din, :dout].set(w).at[din:, dout:].set(w)

    def dbl_b(b):
        return jnp.concatenate([b, b])[None, :]

    def dbl_gru(gp):
        out = {}
        for g in ("z", "r", "h"):
            w1_, b1_ = gp[g + "1"]["w"], gp[g + "1"]["b"]
            w2_, b2_ = gp[g + "2"]["w"], gp[g + "2"]["b"]
            out[g] = (dbl(w1_[:D]), dbl(w1_[D:D + IN_DIM]), dbl_b(b1_),
                      dbl(w2_), dbl_b(b2_))
        return out

    p1, p2 = params["gde1"], params["gde2"]
    genc2 = dbl_gru(params["gru_enc"])
    gdec2 = dbl_gru(params["gru_dec"])
    ow2 = dbl(params["out"]["w"])
    ob2 = dbl_b(params["out"]["b"])

    deg_part = _sc_deg(sidx_deg, jnp.ones((DCH, DEGW), jnp.float32),
                       jnp.zeros((N, DEGW), jnp.float32))
    w1, w2, inv = _tc_env(p1["comp"], p1["basis"], p2["comp"], p2["basis"],
                          deg_part)
    w1d = jax.vmap(dbl)(w1.reshape(R, D, D))
    w2d = jax.vmap(dbl)(w2.reshape(R, D, D))
    root1d, root2d = dbl(p1["root"]), dbl(p2["root"])
    bias1d, bias2d = dbl_b(p1["bias"]), dbl_b(p2["bias"])
    inv_rep = jnp.repeat(inv.reshape(N), NG).reshape(SROWS, 1)

    z = jnp.zeros((SROWS, SW), jnp.float32)
    hr = jnp.zeros((R, SROWS, SW), jnp.float32)
    hroot = jnp.zeros((SROWS, SW), jnp.float32)
    zero_agg = jnp.zeros((NC, SROWS, SW), jnp.float32)

    preds = []
    for step in range(3 * HORIZON):
        enc = step < HORIZON
        wd, rootd, biasd = ((w1d, root1d, bias1d) if enc
                            else (w2d, root2d, bias2d))
        if step == 0:
            agg = zero_agg
        else:
            agg = _sc_agg(hr.reshape(R * SROWS, SW), gidx, sidx, zeros_rows)
        z, hr, hroot = _tc_postpre(z, agg, hroot, inv_rep, biasd, wd, rootd)
        agg = _sc_agg(hr.reshape(R * SROWS, SW), gidx, sidx, zeros_rows)
        nxt_enc = (step + 1) < HORIZON
        wnd, rootnd = (w1d, root1d) if nxt_enc else (w2d, root2d)
        gp2 = genc2 if enc else gdec2
        xt = data_nb[HORIZON - 1 - step] if enc else data_nb[step - HORIZON]
        outs = _tc_gru(z, agg, hroot, inv_rep, biasd, xt, gp2, wnd, rootnd,
                       ow2, ob2, with_pred=not enc)
        if enc:
            z, hr, hroot = outs
        else:
            z, hr, hroot, pred = outs
            preds.append(pred)

    preds = jnp.stack(preds)
    preds = preds.reshape(2 * HORIZON, N, B)
    id_times = jnp.argsort(times[:, 0], stable=True)
    keep = jnp.nonzero(id_times >= HORIZON, size=HORIZON)[0]
    out = preds[keep]
    return jnp.transpose(out, (2, 0, 1))[..., None]


# speedup vs baseline: 35.6655x; 1.0730x over previous
import functools

import jax
import jax.numpy as jnp
from jax import lax
from jax.experimental import pallas as pl
from jax.experimental.pallas import tpu as pltpu
from jax.experimental.pallas import tpu_sc as plsc

N = 1024; E = 16384; D = 64; IN_DIM = 2; OUT_DIM = 1
B = 8; HORIZON = 12; R = 3; NB = 3; GRU_UNITS = 100; ODE_STEPS = 2
DT = 1.0 / ODE_STEPS

NC, NS = 2, 16
EPW = E // (NC * NS)
BD = B * D
GW = 128
NG = BD // GW
RPW = EPW * NG
NBUF = 5
PREF = 3
CHUNK = 128
NCHUNK = RPW // CHUNK
AGG_ROWS = N * NG
ROWS_PER_TILE = AGG_ROWS // NS


def _sc_mesh():
    return plsc.VectorSubcoreMesh(core_axis_name="c", subcore_axis_name="s",
                                  num_cores=NC, num_subcores=NS)


@functools.cache
def _sc_agg_kernel():
    return pl.kernel(
        _sc_agg_body,
        mesh=_sc_mesh(),
        out_type=jax.ShapeDtypeStruct((NC, AGG_ROWS, GW), jnp.float32),
        scratch_types=[
            pltpu.VMEM((NCHUNK, CHUNK), jnp.int32),
            pltpu.VMEM((NCHUNK, CHUNK), jnp.int32),
            [pltpu.VMEM((CHUNK, GW), jnp.float32)] * NBUF,
            pltpu.VMEM_SHARED((AGG_ROWS, GW), jnp.float32),
            [pltpu.SemaphoreType.DMA] * NBUF,
            [pltpu.SemaphoreType.DMA] * NBUF,
        ],
    )


def _sc_agg(hr_flat, gidx, sidx, zeros_rows):
    return _sc_agg_kernel()(hr_flat, gidx, sidx, zeros_rows)


def _sc_agg_body(hr_hbm, gidx_hbm, sidx_hbm, zeros_hbm, out_hbm,
                 gidx_v, sidx_v, bufs, agg_sh, gsem, ssem):
    c = lax.axis_index("c")
    s = lax.axis_index("s")
    pltpu.sync_copy(gidx_hbm.at[c, s], gidx_v)
    pltpu.sync_copy(sidx_hbm.at[c, s], sidx_v)

    def gather(k):
        pltpu.async_copy(hr_hbm.at[gidx_v.at[k]], bufs[k % NBUF],
                         gsem[k % NBUF])

    def gather_wait(k):
        pltpu.make_async_copy(hr_hbm.at[gidx_v.at[k]], bufs[k % NBUF],
                              gsem[k % NBUF]).wait()

    def scatter(k):
        pltpu.async_copy(bufs[k % NBUF], agg_sh.at[sidx_v.at[k]],
                         ssem[k % NBUF], add=True)

    def scatter_wait(k):
        pltpu.make_async_copy(bufs[k % NBUF], agg_sh.at[sidx_v.at[k]],
                              ssem[k % NBUF]).wait()

    for k in range(PREF):
        gather(k)
    row0 = s * ROWS_PER_TILE
    pltpu.sync_copy(zeros_hbm.at[pl.ds(row0, ROWS_PER_TILE)],
                    agg_sh.at[pl.ds(row0, ROWS_PER_TILE)])
    plsc.subcore_barrier()
    waited = -1
    for j in range(NCHUNK):
        if j + PREF < NCHUNK:
            if j - (NBUF - PREF) >= 0:
                waited = j - (NBUF - PREF)
                scatter_wait(waited)
            gather(j + PREF)
        gather_wait(j)
        scatter(j)
    for j in range(waited + 1, NCHUNK):
        scatter_wait(j)
    plsc.subcore_barrier()
    pltpu.sync_copy(agg_sh.at[pl.ds(row0, ROWS_PER_TILE)],
                    out_hbm.at[c, pl.ds(row0, ROWS_PER_TILE)])


DEGW = 128
DCH = 64
DNCH = EPW // DCH
DROWS_PER_TILE = N // NS


@functools.cache
def _sc_deg_kernel():
    return pl.kernel(
        _sc_deg_body,
        mesh=_sc_mesh(),
        out_type=jax.ShapeDtypeStruct((NC, N, DEGW), jnp.float32),
        scratch_types=[
            pltpu.VMEM((DNCH, DCH), jnp.int32),
            pltpu.VMEM((DCH, DEGW), jnp.float32),
            pltpu.VMEM_SHARED((N, DEGW), jnp.float32),
            pltpu.SemaphoreType.DMA,
        ],
    )


def _sc_deg(sidx, ones, zeros):
    return _sc_deg_kernel()(sidx, ones, zeros)


def _sc_deg_body(sidx_hbm, ones_hbm, zeros_hbm, out_hbm, sidx_v, ones_v,
                 deg_sh, sem_s):
    c = lax.axis_index("c")
    s = lax.axis_index("s")
    pltpu.sync_copy(sidx_hbm.at[c, s], sidx_v)
    pltpu.sync_copy(ones_hbm, ones_v)
    row0 = s * DROWS_PER_TILE
    pltpu.sync_copy(zeros_hbm.at[pl.ds(row0, DROWS_PER_TILE)],
                    deg_sh.at[pl.ds(row0, DROWS_PER_TILE)])
    plsc.subcore_barrier()
    for j in range(DNCH):
        pltpu.async_copy(ones_v, deg_sh.at[sidx_v.at[j]], sem_s, add=True).wait()
    plsc.subcore_barrier()
    pltpu.sync_copy(deg_sh.at[pl.ds(row0, DROWS_PER_TILE)],
                    out_hbm.at[c, pl.ds(row0, DROWS_PER_TILE)])


def _env_body(comp1_ref, basis1_ref, comp2_ref, basis2_ref, deg_ref,
              w1_ref, w2_ref, inv_ref):
    w1_ref[...] = comp1_ref[...] @ basis1_ref[...]
    w2_ref[...] = comp2_ref[...] @ basis2_ref[...]
    deg = deg_ref[0, :, 0:1] + deg_ref[1, :, 0:1]
    inv_ref[...] = 1.0 / jnp.maximum(deg, 1.0)


def _tc_env(comp1, basis1, comp2, basis2, deg_part):
    return pl.pallas_call(
        _env_body,
        out_shape=[jax.ShapeDtypeStruct((R, D * D), jnp.float32),
                   jax.ShapeDtypeStruct((R, D * D), jnp.float32),
                   jax.ShapeDtypeStruct((N, 1), jnp.float32)],
    )(comp1, basis1.reshape(NB, D * D), comp2, basis2.reshape(NB, D * D),
      deg_part)


SROWS = N * NG
SW = 2 * D
G2 = 2 * GRU_UNITS
X2 = 2 * IN_DIM
P2 = 2 * OUT_DIM


def _rgcn_update(z, agg_ref, hroot_ref, inv_ref, bias_ref):
    agg = (agg_ref[0] + agg_ref[1]) * inv_ref[...]
    return z + DT * jnp.tanh(agg + hroot_ref[...] + bias_ref[...])


def _proj(z, w_ref, root_ref, hr_ref, hroot_ref):
    for r in range(R):
        hr_ref[r] = z @ w_ref[r]
    hroot_ref[...] = z @ root_ref[...]


def _postpre_body(z_ref, agg_ref, hroot_ref, inv_ref, bias_ref,
                  w_ref, root_ref, zo_ref, hr_ref, hroot_o_ref):
    z = _rgcn_update(z_ref[...], agg_ref, hroot_ref, inv_ref, bias_ref)
    zo_ref[...] = z
    _proj(z, w_ref, root_ref, hr_ref, hroot_o_ref)


BM = 2048
NBLK = SROWS // BM

_rows = lambda i: (i, 0)
_rows3 = lambda i: (0, i, 0)
_full2 = lambda i: (0, 0)
_full3 = lambda i: (0, 0, 0)
_rowsP = lambda i, t: (i, 0)
_rows3P = lambda i, t: (0, i, 0)
_full2P = lambda i, t: (0, 0)
_full3P = lambda i, t: (0, 0, 0)


def _tc_postpre(z, agg, hroot, inv_rep, bias2, w2, root2):
    return pl.pallas_call(
        _postpre_body,
        grid=(NBLK,),
        in_specs=[pl.BlockSpec((BM, SW), _rows),
                  pl.BlockSpec((NC, BM, SW), _rows3),
                  pl.BlockSpec((BM, SW), _rows),
                  pl.BlockSpec((BM, 1), _rows),
                  pl.BlockSpec((1, SW), _full2),
                  pl.BlockSpec((R, SW, SW), _full3),
                  pl.BlockSpec((SW, SW), _full2)],
        out_specs=[pl.BlockSpec((BM, SW), _rows),
                   pl.BlockSpec((R, BM, SW), _rows3),
                   pl.BlockSpec((BM, SW), _rows)],
        out_shape=[jax.ShapeDtypeStruct((SROWS, SW), jnp.float32),
                   jax.ShapeDtypeStruct((R, SROWS, SW), jnp.float32),
                   jax.ShapeDtypeStruct((SROWS, SW), jnp.float32)],
    )(z, agg, hroot, inv_rep, bias2, w2, root2)


def _lin(h, w_ref, b_ref):
    return h @ w_ref[...] + b_ref[...]


def _gru_body(with_pred, t_ref, z_ref, agg_ref, hroot_ref, inv_ref, bias_ref,
              x_ref, wz1h_ref, wz1x_ref, bz1_ref, wz2_ref, bz2_ref,
              wr1h_ref, wr1x_ref, br1_ref, wr2_ref, br2_ref,
              wh1h_ref, wh1x_ref, bh1_ref, wh2_ref, bh2_ref,
              wn_ref, rootn_ref, ow_ref, ob_ref,
              zo_ref, hr_ref, hroot_o_ref, *maybe_pred):
    h = _rgcn_update(z_ref[...], agg_ref, hroot_ref, inv_ref, bias_ref)
    x = x_ref[0]
    def cat_lin(hh, wh_ref, wx_ref, b_ref):
        return hh @ wh_ref[...] + x @ wx_ref[...] + b_ref[...]

    zg = jax.nn.sigmoid(_lin(jnp.tanh(cat_lin(h, wz1h_ref, wz1x_ref, bz1_ref)),
                             wz2_ref, bz2_ref))
    rg = jax.nn.sigmoid(_lin(jnp.tanh(cat_lin(h, wr1h_ref, wr1x_ref, br1_ref)),
                             wr2_ref, br2_ref))
    hn = jnp.tanh(_lin(jnp.tanh(cat_lin(h * rg, wh1h_ref, wh1x_ref, bh1_ref)),
                       wh2_ref, bh2_ref))
    hnew = (1.0 - zg) * hn + zg * h
    zo_ref[...] = hnew
    _proj(hnew, wn_ref, rootn_ref, hr_ref, hroot_o_ref)
    if with_pred:
        maybe_pred[0][...] = hnew @ ow_ref[...] + ob_ref[...]


def _tc_gru(z, agg, hroot, inv_rep, bias2, data_nb, tt, gp2, wn2, rootn2,
            ow2, ob2, with_pred):
    out_shape = [jax.ShapeDtypeStruct((SROWS, SW), jnp.float32),
                 jax.ShapeDtypeStruct((R, SROWS, SW), jnp.float32),
                 jax.ShapeDtypeStruct((SROWS, SW), jnp.float32)]
    out_specs = [pl.BlockSpec((BM, SW), _rowsP),
                 pl.BlockSpec((R, BM, SW), _rows3P),
                 pl.BlockSpec((BM, SW), _rowsP)]
    if with_pred:
        out_shape.append(jax.ShapeDtypeStruct((SROWS, P2), jnp.float32))
        out_specs.append(pl.BlockSpec((BM, P2), _rowsP))
    in_specs = ([pl.BlockSpec((BM, SW), _rowsP),
                 pl.BlockSpec((NC, BM, SW), _rows3P),
                 pl.BlockSpec((BM, SW), _rowsP),
                 pl.BlockSpec((BM, 1), _rowsP),
                 pl.BlockSpec((1, SW), _full2P),
                 pl.BlockSpec((1, BM, X2), lambda i, t_ref: (t_ref[0], i, 0))]
                + [pl.BlockSpec((SW, G2), _full2P),
                   pl.BlockSpec((X2, G2), _full2P),
                   pl.BlockSpec((1, G2), _full2P),
                   pl.BlockSpec((G2, SW), _full2P),
                   pl.BlockSpec((1, SW), _full2P)] * 3
                + [pl.BlockSpec((R, SW, SW), _full3P),
                   pl.BlockSpec((SW, SW), _full2P),
                   pl.BlockSpec((SW, P2), _full2P),
                   pl.BlockSpec((1, P2), _full2P)])
    return pl.pallas_call(
        functools.partial(_gru_body, with_pred),
        grid_spec=pltpu.PrefetchScalarGridSpec(
            num_scalar_prefetch=1,
            grid=(NBLK,),
            in_specs=in_specs,
            out_specs=out_specs,
        ),
        out_shape=out_shape,
    )(tt, z, agg, hroot, inv_rep, bias2, data_nb,
      *gp2["z"], *gp2["r"], *gp2["h"],
      wn2, rootn2, ow2, ob2)


def kernel(x, y, xtime, ytime, params, edge_index, edge_type):
    src, dst = edge_index[0], edge_index[1]
    groups = jnp.arange(NG, dtype=jnp.int32)
    gidx = ((edge_type * N + src)[:, None] * NG + groups).reshape(
        NC, NS, NCHUNK, CHUNK)
    sidx = (dst[:, None] * NG + groups).reshape(NC, NS, NCHUNK, CHUNK)
    sidx_deg = dst.reshape(NC, NS, DNCH, DCH)
    zeros_rows = jnp.zeros((AGG_ROWS, GW), jnp.float32)

    data = jnp.concatenate([x, y], axis=0)
    data_nb = jnp.transpose(data, (0, 2, 1, 3)).reshape(
        2 * HORIZON, N, NG, 2 * IN_DIM).reshape(2 * HORIZON, SROWS, X2)
    times = jnp.concatenate([xtime, ytime], axis=0)

    def dbl(w):
        din, dout = w.shape
        z = jnp.zeros((2 * din, 2 * dout), w.dtype)
        return z.at[:din, :dout].set(w).at[din:, dout:].set(w)

    def dbl_b(b):
        return jnp.concatenate([b, b])[None, :]

    def dbl_gru(gp):
        out = {}
        for g in ("z", "r", "h"):
            w1_, b1_ = gp[g + "1"]["w"], gp[g + "1"]["b"]
            w2_, b2_ = gp[g + "2"]["w"], gp[g + "2"]["b"]
            out[g] = (dbl(w1_[:D]), dbl(w1_[D:D + IN_DIM]), dbl_b(b1_),
                      dbl(w2_), dbl_b(b2_))
        return out

    p1, p2 = params["gde1"], params["gde2"]
    genc2 = dbl_gru(params["gru_enc"])
    gdec2 = dbl_gru(params["gru_dec"])
    ow2 = dbl(params["out"]["w"])
    ob2 = dbl_b(params["out"]["b"])

    deg_part = _sc_deg(sidx_deg, jnp.ones((DCH, DEGW), jnp.float32),
                       jnp.zeros((N, DEGW), jnp.float32))
    w1, w2, inv = _tc_env(p1["comp"], p1["basis"], p2["comp"], p2["basis"],
                          deg_part)
    w1d = jax.vmap(dbl)(w1.reshape(R, D, D))
    w2d = jax.vmap(dbl)(w2.reshape(R, D, D))
    root1d, root2d = dbl(p1["root"]), dbl(p2["root"])
    bias1d, bias2d = dbl_b(p1["bias"]), dbl_b(p2["bias"])
    inv_rep = jnp.repeat(inv.reshape(N), NG).reshape(SROWS, 1)

    z = jnp.zeros((SROWS, SW), jnp.float32)
    hr = jnp.zeros((R, SROWS, SW), jnp.float32)
    hroot = jnp.zeros((SROWS, SW), jnp.float32)
    zero_agg = jnp.zeros((NC, SROWS, SW), jnp.float32)

    preds = []
    for step in range(3 * HORIZON):
        enc = step < HORIZON
        wd, rootd, biasd = ((w1d, root1d, bias1d) if enc
                            else (w2d, root2d, bias2d))
        if step == 0:
            agg = zero_agg
        else:
            agg = _sc_agg(hr.reshape(R * SROWS, SW), gidx, sidx, zeros_rows)
        z, hr, hroot = _tc_postpre(z, agg, hroot, inv_rep, biasd, wd, rootd)
        agg = _sc_agg(hr.reshape(R * SROWS, SW), gidx, sidx, zeros_rows)
        nxt_enc = (step + 1) < HORIZON
        wnd, rootnd = (w1d, root1d) if nxt_enc else (w2d, root2d)
        gp2 = genc2 if enc else gdec2
        tt = jnp.array([HORIZON - 1 - step if enc else step - HORIZON],
                       jnp.int32)
        outs = _tc_gru(z, agg, hroot, inv_rep, biasd, data_nb, tt, gp2,
                       wnd, rootnd, ow2, ob2, with_pred=not enc)
        if enc:
            z, hr, hroot = outs
        else:
            z, hr, hroot, pred = outs
            preds.append(pred)

    preds = jnp.stack(preds)
    preds = preds.reshape(2 * HORIZON, N, B)
    id_times = jnp.argsort(times[:, 0], stable=True)
    keep = jnp.nonzero(id_times >= HORIZON, size=HORIZON)[0]
    out = preds[keep]
    return jnp.transpose(out, (2, 0, 1))[..., None]


# speedup vs baseline: 36.6303x; 1.0270x over previous
import functools

import jax
import jax.numpy as jnp
from jax import lax
from jax.experimental import pallas as pl
from jax.experimental.pallas import tpu as pltpu
from jax.experimental.pallas import tpu_sc as plsc

N = 1024; E = 16384; D = 64; IN_DIM = 2; OUT_DIM = 1
B = 8; HORIZON = 12; R = 3; NB = 3; GRU_UNITS = 100; ODE_STEPS = 2
DT = 1.0 / ODE_STEPS

NC, NS = 2, 16
EPW = E // (NC * NS)
BD = B * D
GW = 128
NG = BD // GW
RPW = EPW * NG
NBUF = 5
PREF = 3
CHUNK = 128
NCHUNK = RPW // CHUNK
AGG_ROWS = N * NG
ROWS_PER_TILE = AGG_ROWS // NS


def _sc_mesh():
    return plsc.VectorSubcoreMesh(core_axis_name="c", subcore_axis_name="s",
                                  num_cores=NC, num_subcores=NS)


@functools.cache
def _sc_agg_kernel():
    return pl.kernel(
        _sc_agg_body,
        mesh=_sc_mesh(),
        out_type=jax.ShapeDtypeStruct((NC, AGG_ROWS, GW), jnp.float32),
        scratch_types=[
            pltpu.VMEM((NCHUNK, CHUNK), jnp.int32),
            pltpu.VMEM((NCHUNK, CHUNK), jnp.int32),
            [pltpu.VMEM((CHUNK, GW), jnp.float32)] * NBUF,
            pltpu.VMEM_SHARED((AGG_ROWS, GW), jnp.float32),
            [pltpu.SemaphoreType.DMA] * NBUF,
            [pltpu.SemaphoreType.DMA] * NBUF,
        ],
    )


def _sc_agg(hr_flat, gidx, sidx, zeros_rows):
    return _sc_agg_kernel()(hr_flat, gidx, sidx, zeros_rows)


def _sc_agg_body(hr_hbm, gidx_hbm, sidx_hbm, zeros_hbm, out_hbm,
                 gidx_v, sidx_v, bufs, agg_sh, gsem, ssem):
    c = lax.axis_index("c")
    s = lax.axis_index("s")
    pltpu.sync_copy(gidx_hbm.at[c, s], gidx_v)
    pltpu.sync_copy(sidx_hbm.at[c, s], sidx_v)

    def gather(k):
        pltpu.async_copy(hr_hbm.at[gidx_v.at[k]], bufs[k % NBUF],
                         gsem[k % NBUF])

    def gather_wait(k):
        pltpu.make_async_copy(hr_hbm.at[gidx_v.at[k]], bufs[k % NBUF],
                              gsem[k % NBUF]).wait()

    def scatter(k):
        pltpu.async_copy(bufs[k % NBUF], agg_sh.at[sidx_v.at[k]],
                         ssem[k % NBUF], add=True)

    def scatter_wait(k):
        pltpu.make_async_copy(bufs[k % NBUF], agg_sh.at[sidx_v.at[k]],
                              ssem[k % NBUF]).wait()

    for k in range(PREF):
        gather(k)
    row0 = s * ROWS_PER_TILE
    pltpu.sync_copy(zeros_hbm.at[pl.ds(row0, ROWS_PER_TILE)],
                    agg_sh.at[pl.ds(row0, ROWS_PER_TILE)])
    plsc.subcore_barrier()
    waited = -1
    for j in range(NCHUNK):
        if j + PREF < NCHUNK:
            if j - (NBUF - PREF) >= 0:
                waited = j - (NBUF - PREF)
                scatter_wait(waited)
            gather(j + PREF)
        gather_wait(j)
        scatter(j)
    for j in range(waited + 1, NCHUNK):
        scatter_wait(j)
    plsc.subcore_barrier()
    pltpu.sync_copy(agg_sh.at[pl.ds(row0, ROWS_PER_TILE)],
                    out_hbm.at[c, pl.ds(row0, ROWS_PER_TILE)])


DEGW = 128
DCH = 64
DNCH = EPW // DCH
DROWS_PER_TILE = N // NS


@functools.cache
def _sc_deg_kernel():
    return pl.kernel(
        _sc_deg_body,
        mesh=_sc_mesh(),
        out_type=jax.ShapeDtypeStruct((NC, N, DEGW), jnp.float32),
        scratch_types=[
            pltpu.VMEM((DNCH, DCH), jnp.int32),
            pltpu.VMEM((DCH, DEGW), jnp.float32),
            pltpu.VMEM_SHARED((N, DEGW), jnp.float32),
            pltpu.SemaphoreType.DMA,
        ],
    )


def _sc_deg(sidx, ones, zeros):
    return _sc_deg_kernel()(sidx, ones, zeros)


def _sc_deg_body(sidx_hbm, ones_hbm, zeros_hbm, out_hbm, sidx_v, ones_v,
                 deg_sh, sem_s):
    c = lax.axis_index("c")
    s = lax.axis_index("s")
    pltpu.sync_copy(sidx_hbm.at[c, s], sidx_v)
    pltpu.sync_copy(ones_hbm, ones_v)
    row0 = s * DROWS_PER_TILE
    pltpu.sync_copy(zeros_hbm.at[pl.ds(row0, DROWS_PER_TILE)],
                    deg_sh.at[pl.ds(row0, DROWS_PER_TILE)])
    plsc.subcore_barrier()
    for j in range(DNCH):
        pltpu.async_copy(ones_v, deg_sh.at[sidx_v.at[j]], sem_s, add=True).wait()
    plsc.subcore_barrier()
    pltpu.sync_copy(deg_sh.at[pl.ds(row0, DROWS_PER_TILE)],
                    out_hbm.at[c, pl.ds(row0, DROWS_PER_TILE)])


def _env_body(comp1_ref, basis1_ref, comp2_ref, basis2_ref, deg_ref,
              w1_ref, w2_ref, inv_ref):
    w1_ref[...] = comp1_ref[...] @ basis1_ref[...]
    w2_ref[...] = comp2_ref[...] @ basis2_ref[...]
    deg = deg_ref[0, :, 0:1] + deg_ref[1, :, 0:1]
    inv_ref[...] = 1.0 / jnp.maximum(deg, 1.0)


def _tc_env(comp1, basis1, comp2, basis2, deg_part):
    return pl.pallas_call(
        _env_body,
        out_shape=[jax.ShapeDtypeStruct((R, D * D), jnp.float32),
                   jax.ShapeDtypeStruct((R, D * D), jnp.float32),
                   jax.ShapeDtypeStruct((N, 1), jnp.float32)],
    )(comp1, basis1.reshape(NB, D * D), comp2, basis2.reshape(NB, D * D),
      deg_part)


SROWS = N * NG
SW = 2 * D
G2 = 2 * GRU_UNITS
X2 = 2 * IN_DIM
P2 = 2 * OUT_DIM


def _rgcn_update(z, agg_ref, inv_ref, bias_ref, root_ref):
    agg = (agg_ref[0] + agg_ref[1]) * inv_ref[...]
    return z + DT * jnp.tanh(agg + z @ root_ref[...] + bias_ref[...])


def _proj(z, w_ref, hr_ref):
    for r in range(R):
        hr_ref[r] = z @ w_ref[r]


def _postpre_body(z_ref, agg_ref, inv_ref, bias_ref, root_ref,
                  w_ref, zo_ref, hr_ref):
    z = _rgcn_update(z_ref[...], agg_ref, inv_ref, bias_ref, root_ref)
    zo_ref[...] = z
    _proj(z, w_ref, hr_ref)


BM = 2048
NBLK = SROWS // BM

_rows = lambda i: (i, 0)
_rows3 = lambda i: (0, i, 0)
_full2 = lambda i: (0, 0)
_full3 = lambda i: (0, 0, 0)
_rowsP = lambda i, t: (i, 0)
_rows3P = lambda i, t: (0, i, 0)
_full2P = lambda i, t: (0, 0)
_full3P = lambda i, t: (0, 0, 0)


def _tc_postpre(z, agg, inv_rep, bias2, root2, w2):
    return pl.pallas_call(
        _postpre_body,
        grid=(NBLK,),
        in_specs=[pl.BlockSpec((BM, SW), _rows),
                  pl.BlockSpec((NC, BM, SW), _rows3),
                  pl.BlockSpec((BM, 1), _rows),
                  pl.BlockSpec((1, SW), _full2),
                  pl.BlockSpec((SW, SW), _full2),
                  pl.BlockSpec((R, SW, SW), _full3)],
        out_specs=[pl.BlockSpec((BM, SW), _rows),
                   pl.BlockSpec((R, BM, SW), _rows3)],
        out_shape=[jax.ShapeDtypeStruct((SROWS, SW), jnp.float32),
                   jax.ShapeDtypeStruct((R, SROWS, SW), jnp.float32)],
    )(z, agg, inv_rep, bias2, root2, w2)


def _lin(h, w_ref, b_ref):
    return h @ w_ref[...] + b_ref[...]


def _gru_body(with_pred, t_ref, z_ref, agg_ref, inv_ref, bias_ref, root_ref,
              x_ref, wz1h_ref, wz1x_ref, bz1_ref, wz2_ref, bz2_ref,
              wr1h_ref, wr1x_ref, br1_ref, wr2_ref, br2_ref,
              wh1h_ref, wh1x_ref, bh1_ref, wh2_ref, bh2_ref,
              wn_ref, ow_ref, ob_ref,
              zo_ref, hr_ref, *maybe_pred):
    h = _rgcn_update(z_ref[...], agg_ref, inv_ref, bias_ref, root_ref)
    x = x_ref[0]
    def cat_lin(hh, wh_ref, wx_ref, b_ref):
        return hh @ wh_ref[...] + x @ wx_ref[...] + b_ref[...]

    zg = jax.nn.sigmoid(_lin(jnp.tanh(cat_lin(h, wz1h_ref, wz1x_ref, bz1_ref)),
                             wz2_ref, bz2_ref))
    rg = jax.nn.sigmoid(_lin(jnp.tanh(cat_lin(h, wr1h_ref, wr1x_ref, br1_ref)),
                             wr2_ref, br2_ref))
    hn = jnp.tanh(_lin(jnp.tanh(cat_lin(h * rg, wh1h_ref, wh1x_ref, bh1_ref)),
                       wh2_ref, bh2_ref))
    hnew = (1.0 - zg) * hn + zg * h
    zo_ref[...] = hnew
    _proj(hnew, wn_ref, hr_ref)
    if with_pred:
        maybe_pred[0][...] = hnew @ ow_ref[...] + ob_ref[...]


def _tc_gru(z, agg, inv_rep, bias2, root2, data_nb, tt, gp2, wn2,
            ow2, ob2, with_pred):
    out_shape = [jax.ShapeDtypeStruct((SROWS, SW), jnp.float32),
                 jax.ShapeDtypeStruct((R, SROWS, SW), jnp.float32)]
    out_specs = [pl.BlockSpec((BM, SW), _rowsP),
                 pl.BlockSpec((R, BM, SW), _rows3P)]
    if with_pred:
        out_shape.append(jax.ShapeDtypeStruct((SROWS, P2), jnp.float32))
        out_specs.append(pl.BlockSpec((BM, P2), _rowsP))
    in_specs = ([pl.BlockSpec((BM, SW), _rowsP),
                 pl.BlockSpec((NC, BM, SW), _rows3P),
                 pl.BlockSpec((BM, 1), _rowsP),
                 pl.BlockSpec((1, SW), _full2P),
                 pl.BlockSpec((SW, SW), _full2P),
                 pl.BlockSpec((1, BM, X2), lambda i, t_ref: (t_ref[0], i, 0))]
                + [pl.BlockSpec((SW, G2), _full2P),
                   pl.BlockSpec((X2, G2), _full2P),
                   pl.BlockSpec((1, G2), _full2P),
                   pl.BlockSpec((G2, SW), _full2P),
                   pl.BlockSpec((1, SW), _full2P)] * 3
                + [pl.BlockSpec((R, SW, SW), _full3P),
                   pl.BlockSpec((SW, P2), _full2P),
                   pl.BlockSpec((1, P2), _full2P)])
    return pl.pallas_call(
        functools.partial(_gru_body, with_pred),
        grid_spec=pltpu.PrefetchScalarGridSpec(
            num_scalar_prefetch=1,
            grid=(NBLK,),
            in_specs=in_specs,
            out_specs=out_specs,
        ),
        out_shape=out_shape,
    )(tt, z, agg, inv_rep, bias2, root2, data_nb,
      *gp2["z"], *gp2["r"], *gp2["h"],
      wn2, ow2, ob2)


def kernel(x, y, xtime, ytime, params, edge_index, edge_type):
    src, dst = edge_index[0], edge_index[1]
    groups = jnp.arange(NG, dtype=jnp.int32)
    gidx = ((edge_type * N + src)[:, None] * NG + groups).reshape(
        NC, NS, NCHUNK, CHUNK)
    sidx = (dst[:, None] * NG + groups).reshape(NC, NS, NCHUNK, CHUNK)
    sidx_deg = dst.reshape(NC, NS, DNCH, DCH)
    zeros_rows = jnp.zeros((AGG_ROWS, GW), jnp.float32)

    data = jnp.concatenate([x, y], axis=0)
    data_nb = jnp.transpose(data, (0, 2, 1, 3)).reshape(
        2 * HORIZON, N, NG, 2 * IN_DIM).reshape(2 * HORIZON, SROWS, X2)
    times = jnp.concatenate([xtime, ytime], axis=0)

    def dbl(w):
        din, dout = w.shape
        z = jnp.zeros((2 * din, 2 * dout), w.dtype)
        return z.at[:din, :dout].set(w).at[din:, dout:].set(w)

    def dbl_b(b):
        return jnp.concatenate([b, b])[None, :]

    def dbl_gru(gp):
        out = {}
        for g in ("z", "r", "h"):
            w1_, b1_ = gp[g + "1"]["w"], gp[g + "1"]["b"]
            w2_, b2_ = gp[g + "2"]["w"], gp[g + "2"]["b"]
            out[g] = (dbl(w1_[:D]), dbl(w1_[D:D + IN_DIM]), dbl_b(b1_),
                      dbl(w2_), dbl_b(b2_))
        return out

    p1, p2 = params["gde1"], params["gde2"]
    genc2 = dbl_gru(params["gru_enc"])
    gdec2 = dbl_gru(params["gru_dec"])
    ow2 = dbl(params["out"]["w"])
    ob2 = dbl_b(params["out"]["b"])

    deg_part = _sc_deg(sidx_deg, jnp.ones((DCH, DEGW), jnp.float32),
                       jnp.zeros((N, DEGW), jnp.float32))
    w1, w2, inv = _tc_env(p1["comp"], p1["basis"], p2["comp"], p2["basis"],
                          deg_part)
    w1d = jax.vmap(dbl)(w1.reshape(R, D, D))
    w2d = jax.vmap(dbl)(w2.reshape(R, D, D))
    root1d, root2d = dbl(p1["root"]), dbl(p2["root"])
    bias1d, bias2d = dbl_b(p1["bias"]), dbl_b(p2["bias"])
    inv_rep = jnp.repeat(inv.reshape(N), NG).reshape(SROWS, 1)

    z = jnp.zeros((SROWS, SW), jnp.float32)
    hr = jnp.zeros((R, SROWS, SW), jnp.float32)
    zero_agg = jnp.zeros((NC, SROWS, SW), jnp.float32)

    preds = []
    for step in range(3 * HORIZON):
        enc = step < HORIZON
        wd, rootd, biasd = ((w1d, root1d, bias1d) if enc
                            else (w2d, root2d, bias2d))
        if step == 0:
            agg = zero_agg
        else:
            agg = _sc_agg(hr.reshape(R * SROWS, SW), gidx, sidx, zeros_rows)
        z, hr = _tc_postpre(z, agg, inv_rep, biasd, rootd, wd)
        agg = _sc_agg(hr.reshape(R * SROWS, SW), gidx, sidx, zeros_rows)
        nxt_enc = (step + 1) < HORIZON
        wnd = w1d if nxt_enc else w2d
        gp2 = genc2 if enc else gdec2
        tt = jnp.array([HORIZON - 1 - step if enc else step - HORIZON],
                       jnp.int32)
        outs = _tc_gru(z, agg, inv_rep, biasd, rootd, data_nb, tt, gp2,
                       wnd, ow2, ob2, with_pred=not enc)
        if enc:
            z, hr = outs
        else:
            z, hr, pred = outs
            preds.append(pred)

    preds = jnp.stack(preds)
    preds = preds.reshape(2 * HORIZON, N, B)
    id_times = jnp.argsort(times[:, 0], stable=True)
    keep = jnp.nonzero(id_times >= HORIZON, size=HORIZON)[0]
    out = preds[keep]
    return jnp.transpose(out, (2, 0, 1))[..., None]
